# Initial kernel scaffold; baseline (speedup 1.0000x reference)
import jax
import jax.numpy as jnp
from jax.experimental import pallas as pl


def kernel(obj_feature, rel_feature, edges_index, params):
    raise NotImplementedError("write your pallas kernel here")


# speedup vs baseline: 2.4987x; 2.4987x over previous
import functools

import jax
import jax.numpy as jnp
from jax import lax
from jax.experimental import pallas as pl
from jax.experimental.pallas import tpu as pltpu
from jax.experimental.pallas import tpu_sc as plsc

N = 10000
E = 160000
D = 128
R = 26

BE = 2000
GE = E // BE
BN = 2000
GN = N // BN

CH = 128
NCH = E // CH
NW = 32
TPW = (NCH + NW - 1) // NW
NS = 16
RPT = 640
N_PAD = NS * RPT


def _full(shape):
    return pl.BlockSpec(shape, lambda i: tuple(0 for _ in shape))


def _mm_stats_body(x_ref, w_ref, b_ref, y_ref, ssum_ref, ssq_ref):
    y = jnp.dot(x_ref[...], w_ref[...], preferred_element_type=jnp.float32)
    y = y + b_ref[...]
    y_ref[...] = y

    @pl.when(pl.program_id(0) == 0)
    def _():
        ssum_ref[...] = jnp.zeros_like(ssum_ref)
        ssq_ref[...] = jnp.zeros_like(ssq_ref)

    ssum_ref[...] += jnp.sum(y, axis=0, keepdims=True)
    ssq_ref[...] += jnp.sum(y * y, axis=0, keepdims=True)


def _mm_stats(x, w, b):
    k_in = x.shape[1]
    k_out = w.shape[1]
    return pl.pallas_call(
        _mm_stats_body,
        grid=(GE,),
        in_specs=[
            pl.BlockSpec((BE, k_in), lambda i: (i, 0)),
            _full((k_in, k_out)),
            _full((1, k_out)),
        ],
        out_specs=[
            pl.BlockSpec((BE, k_out), lambda i: (i, 0)),
            pl.BlockSpec((1, k_out), lambda i: (0, 0)),
            pl.BlockSpec((1, k_out), lambda i: (0, 0)),
        ],
        out_shape=[
            jax.ShapeDtypeStruct((E, k_out), jnp.float32),
            jax.ShapeDtypeStruct((1, k_out), jnp.float32),
            jax.ShapeDtypeStruct((1, k_out), jnp.float32),
        ],
    )(x, w, b)


def _affine_relu_mm_stats_body(x_ref, s_ref, t_ref, w_ref, b_ref,
                               y_ref, ssum_ref, ssq_ref):
    h = jax.nn.relu(x_ref[...] * s_ref[...] + t_ref[...])
    y = jnp.dot(h, w_ref[...], preferred_element_type=jnp.float32) + b_ref[...]
    y_ref[...] = y

    @pl.when(pl.program_id(0) == 0)
    def _():
        ssum_ref[...] = jnp.zeros_like(ssum_ref)
        ssq_ref[...] = jnp.zeros_like(ssq_ref)

    ssum_ref[...] += jnp.sum(y, axis=0, keepdims=True)
    ssq_ref[...] += jnp.sum(y * y, axis=0, keepdims=True)


def _affine_relu_mm_stats(x, s, t, w, b):
    k_in = x.shape[1]
    k_out = w.shape[1]
    return pl.pallas_call(
        _affine_relu_mm_stats_body,
        grid=(GE,),
        in_specs=[
            pl.BlockSpec((BE, k_in), lambda i: (i, 0)),
            _full((1, k_in)),
            _full((1, k_in)),
            _full((k_in, k_out)),
            _full((1, k_out)),
        ],
        out_specs=[
            pl.BlockSpec((BE, k_out), lambda i: (i, 0)),
            pl.BlockSpec((1, k_out), lambda i: (0, 0)),
            pl.BlockSpec((1, k_out), lambda i: (0, 0)),
        ],
        out_shape=[
            jax.ShapeDtypeStruct((E, k_out), jnp.float32),
            jax.ShapeDtypeStruct((1, k_out), jnp.float32),
            jax.ShapeDtypeStruct((1, k_out), jnp.float32),
        ],
    )(x, s, t, w, b)


def _enc_finish_body(y_ref, s_ref, t_ref, w_ref, b_ref, ef_ref, ewb_ref):
    ef = jax.nn.relu(y_ref[...] * s_ref[...] + t_ref[...])
    ef_ref[...] = ef
    ewb_ref[...] = (
        jnp.dot(ef, w_ref[...], preferred_element_type=jnp.float32) + b_ref[...])


def _enc_finish(y2, s, t, wb0, b1_0):
    return pl.pallas_call(
        _enc_finish_body,
        grid=(GE,),
        in_specs=[
            pl.BlockSpec((BE, D), lambda i: (i, 0)),
            _full((1, D)),
            _full((1, D)),
            _full((D, D)),
            _full((1, D)),
        ],
        out_specs=[
            pl.BlockSpec((BE, D), lambda i: (i, 0)),
            pl.BlockSpec((BE, D), lambda i: (i, 0)),
        ],
        out_shape=[
            jax.ShapeDtypeStruct((E, D), jnp.float32),
            jax.ShapeDtypeStruct((E, D), jnp.float32),
        ],
    )(y2, s, t, wb0, b1_0)


def _layer0_edge_body(g_ref, ewb_ref, w2ac_ref, cmsg_ref, w2b_ref, b2b_ref,
                      wb1_ref, b11_ref, msg_ref, ewb1_ref):
    h1 = jax.nn.relu(g_ref[...] + ewb_ref[...])
    msg_ref[...] = (
        jnp.dot(h1, w2ac_ref[...], preferred_element_type=jnp.float32)
        + cmsg_ref[...])
    e1 = jax.nn.relu(
        jnp.dot(h1, w2b_ref[...], preferred_element_type=jnp.float32)
        + b2b_ref[...])
    ewb1_ref[...] = (
        jnp.dot(e1, wb1_ref[...], preferred_element_type=jnp.float32)
        + b11_ref[...])


def _layer0_edge(g0, ewb0, w2ac, cmsg, w2b, b2b, wb1, b11):
    return pl.pallas_call(
        _layer0_edge_body,
        grid=(GE,),
        in_specs=[
            pl.BlockSpec((BE, D), lambda i: (i, 0)),
            pl.BlockSpec((BE, D), lambda i: (i, 0)),
            _full((D, D)),
            _full((1, D)),
            _full((D, D)),
            _full((1, D)),
            _full((D, D)),
            _full((1, D)),
        ],
        out_specs=[
            pl.BlockSpec((BE, D), lambda i: (i, 0)),
            pl.BlockSpec((BE, D), lambda i: (i, 0)),
        ],
        out_shape=[
            jax.ShapeDtypeStruct((E, D), jnp.float32),
            jax.ShapeDtypeStruct((E, D), jnp.float32),
        ],
    )(g0, ewb0, w2ac, cmsg, w2b, b2b, wb1, b11)


def _node_mm_body(x_ref, wa_ref, wc_ref, xa_ref, xc_ref):
    x = x_ref[...]
    xa_ref[...] = jnp.dot(x, wa_ref[...], preferred_element_type=jnp.float32)
    xc_ref[...] = jnp.dot(x, wc_ref[...], preferred_element_type=jnp.float32)


def _node_mm(x, wa, wc):
    return pl.pallas_call(
        _node_mm_body,
        grid=(GN,),
        in_specs=[
            pl.BlockSpec((BN, D), lambda i: (i, 0)),
            _full((D, D)),
            _full((D, D)),
        ],
        out_specs=[
            pl.BlockSpec((BN, D), lambda i: (i, 0)),
            pl.BlockSpec((BN, D), lambda i: (i, 0)),
        ],
        out_shape=[
            jax.ShapeDtypeStruct((N, D), jnp.float32),
            jax.ShapeDtypeStruct((N, D), jnp.float32),
        ],
    )(x, wa, wc)


def _node_update_body(aggp_ref, degp_ref, x_ref, w1_ref, b1_ref, w2_ref,
                      b2_ref, wa_ref, wc_ref, xa_ref, xc_ref):
    deg = jnp.maximum(degp_ref[0, :, 0:1] + degp_ref[1, :, 0:1], 1.0)
    agg = (aggp_ref[0] + aggp_ref[1]) / deg
    h2 = jax.nn.relu(
        jnp.dot(agg, w1_ref[...], preferred_element_type=jnp.float32)
        + b1_ref[...])
    xn = x_ref[...] + (
        jnp.dot(h2, w2_ref[...], preferred_element_type=jnp.float32)
        + b2_ref[...])
    xn = jax.nn.relu(xn)
    xa_ref[...] = jnp.dot(xn, wa_ref[...], preferred_element_type=jnp.float32)
    xc_ref[...] = jnp.dot(xn, wc_ref[...], preferred_element_type=jnp.float32)


def _node_update(aggp, degp, x, w1, b1, w2, b2, wa, wc):
    return pl.pallas_call(
        _node_update_body,
        grid=(GN,),
        in_specs=[
            pl.BlockSpec((2, BN, D), lambda i: (0, i, 0)),
            pl.BlockSpec((2, BN, D), lambda i: (0, i, 0)),
            pl.BlockSpec((BN, D), lambda i: (i, 0)),
            _full((D, D)),
            _full((1, D)),
            _full((D, D)),
            _full((1, D)),
            _full((D, D)),
            _full((D, D)),
        ],
        out_specs=[
            pl.BlockSpec((BN, D), lambda i: (i, 0)),
            pl.BlockSpec((BN, D), lambda i: (i, 0)),
        ],
        out_shape=[
            jax.ShapeDtypeStruct((N, D), jnp.float32),
            jax.ShapeDtypeStruct((N, D), jnp.float32),
        ],
    )(aggp, degp, x, w1, b1, w2, b2, wa, wc)


def _layer1_edge_body(g_ref, ewb_ref, w2b_ref, b2b_ref, cw_ref, cb_ref,
                      ef_ref, cpre_ref, ssum_ref, ssq_ref):
    h1 = jax.nn.relu(g_ref[...] + ewb_ref[...])
    ef = (jnp.dot(h1, w2b_ref[...], preferred_element_type=jnp.float32)
          + b2b_ref[...])
    ef_ref[...] = ef
    cpre = (jnp.dot(ef, cw_ref[...], preferred_element_type=jnp.float32)
            + cb_ref[...])
    cpre_ref[...] = cpre

    @pl.when(pl.program_id(0) == 0)
    def _():
        ssum_ref[...] = jnp.zeros_like(ssum_ref)
        ssq_ref[...] = jnp.zeros_like(ssq_ref)

    ssum_ref[...] += jnp.sum(cpre, axis=0, keepdims=True)
    ssq_ref[...] += jnp.sum(cpre * cpre, axis=0, keepdims=True)


def _layer1_edge(g1, ewb1, w2b, b2b, cw1, cb1):
    hc = cw1.shape[1]
    return pl.pallas_call(
        _layer1_edge_body,
        grid=(GE,),
        in_specs=[
            pl.BlockSpec((BE, D), lambda i: (i, 0)),
            pl.BlockSpec((BE, D), lambda i: (i, 0)),
            _full((D, D)),
            _full((1, D)),
            _full((D, hc)),
            _full((1, hc)),
        ],
        out_specs=[
            pl.BlockSpec((BE, D), lambda i: (i, 0)),
            pl.BlockSpec((BE, hc), lambda i: (i, 0)),
            pl.BlockSpec((1, hc), lambda i: (0, 0)),
            pl.BlockSpec((1, hc), lambda i: (0, 0)),
        ],
        out_shape=[
            jax.ShapeDtypeStruct((E, D), jnp.float32),
            jax.ShapeDtypeStruct((E, hc), jnp.float32),
            jax.ShapeDtypeStruct((1, hc), jnp.float32),
            jax.ShapeDtypeStruct((1, hc), jnp.float32),
        ],
    )(g1, ewb1, w2b, b2b, cw1, cb1)


def _cls_finish_body(c_ref, s_ref, t_ref, w_ref, b_ref, out_ref):
    c = jax.nn.relu(c_ref[...] * s_ref[...] + t_ref[...])
    logits = (jnp.dot(c, w_ref[...], preferred_element_type=jnp.float32)
              + b_ref[...])
    out_ref[...] = jax.nn.sigmoid(logits)


def _cls_finish(cpre, s, t, w, b):
    hc = cpre.shape[1]
    return pl.pallas_call(
        _cls_finish_body,
        grid=(GE,),
        in_specs=[
            pl.BlockSpec((BE, hc), lambda i: (i, 0)),
            _full((1, hc)),
            _full((1, hc)),
            _full((hc, R)),
            _full((1, R)),
        ],
        out_specs=pl.BlockSpec((BE, R), lambda i: (i, 0)),
        out_shape=jax.ShapeDtypeStruct((E, R), jnp.float32),
    )(cpre, s, t, w, b)


def _sc_gather_add(table_a, table_b, idx_a, idx_b):
    mesh = plsc.VectorSubcoreMesh(core_axis_name="c", subcore_axis_name="s")

    @functools.partial(
        pl.kernel,
        mesh=mesh,
        out_type=jax.ShapeDtypeStruct((E, D), jnp.float32),
        scratch_types=[
            pltpu.VMEM((CH,), jnp.int32),
            pltpu.VMEM((CH,), jnp.int32),
            pltpu.VMEM((CH, D), jnp.float32),
            pltpu.VMEM((CH, D), jnp.float32),
            pltpu.SemaphoreType.DMA,
            pltpu.SemaphoreType.DMA,
        ],
    )
    def k(ta_h, tb_h, ia_h, ib_h, out_h, iva, ivb, ra, rb, sa, sb):
        wid = lax.axis_index("s") * 2 + lax.axis_index("c")

        def step(t, carry):
            c = t * NW + wid

            @pl.when(c < NCH)
            def _():
                base = pl.multiple_of(c * CH, CH)
                pltpu.sync_copy(ia_h.at[pl.ds(base, CH)], iva)
                pltpu.sync_copy(ib_h.at[pl.ds(base, CH)], ivb)
                cpa = pltpu.async_copy(ta_h.at[iva], ra, sa)
                cpb = pltpu.async_copy(tb_h.at[ivb], rb, sb)
                cpa.wait()
                cpb.wait()

                def addrow(r, cc):
                    for j in range(D // 16):
                        sl = pl.ds(j * 16, 16)
                        ra[r, sl] = ra[r, sl] + rb[r, sl]
                    return cc

                lax.fori_loop(0, CH, addrow, 0)
                pltpu.sync_copy(ra, out_h.at[pl.ds(base, CH)])

            return carry

        lax.fori_loop(0, TPW, step, 0)

    return k(table_a, table_b, idx_a, idx_b)


def _sc_scatter_msg(msg, dst):
    mesh = plsc.VectorSubcoreMesh(core_axis_name="c", subcore_axis_name="s")

    @functools.partial(
        pl.kernel,
        mesh=mesh,
        out_type=jax.ShapeDtypeStruct((2 * N_PAD, D), jnp.float32),
        scratch_types=[
            pltpu.VMEM((CH,), jnp.int32),
            pltpu.VMEM((CH, D), jnp.float32),
            pltpu.VMEM_SHARED((N_PAD, D), jnp.float32),
        ],
    )
    def k(msg_h, dst_h, agg_h, idxv, rows, acc_s):
        cid = lax.axis_index("c")
        sid = lax.axis_index("s")
        wid = sid * 2 + cid

        def zrow(r, cc):
            for j in range(D // 16):
                rows[r, pl.ds(j * 16, 16)] = jnp.zeros((16,), jnp.float32)
            return cc

        lax.fori_loop(0, CH, zrow, 0)

        row0 = pl.multiple_of(sid * RPT, CH)
        for off in range(0, RPT, CH):
            pltpu.sync_copy(rows, acc_s.at[pl.ds(row0 + off, CH)])

        plsc.subcore_barrier()

        def step(t, carry):
            c = t * NW + wid

            @pl.when(c < NCH)
            def _():
                base = pl.multiple_of(c * CH, CH)
                pltpu.sync_copy(dst_h.at[pl.ds(base, CH)], idxv)
                pltpu.sync_copy(msg_h.at[pl.ds(base, CH)], rows)
                pltpu.sync_copy(rows, acc_s.at[idxv], add=True)

            return carry

        lax.fori_loop(0, TPW, step, 0)

        plsc.subcore_barrier()

        out_row0 = pl.multiple_of(cid * N_PAD + sid * RPT, CH)
        for off in range(0, RPT, CH):
            pltpu.sync_copy(acc_s.at[pl.ds(row0 + off, CH)], rows)
            pltpu.sync_copy(rows, agg_h.at[pl.ds(out_row0 + off, CH)])

    return k(msg, dst)


def _sc_scatter_ones(dst):
    mesh = plsc.VectorSubcoreMesh(core_axis_name="c", subcore_axis_name="s")

    @functools.partial(
        pl.kernel,
        mesh=mesh,
        out_type=jax.ShapeDtypeStruct((2 * N_PAD, D), jnp.float32),
        scratch_types=[
            pltpu.VMEM((CH,), jnp.int32),
            pltpu.VMEM((CH, D), jnp.float32),
            pltpu.VMEM((CH, D), jnp.float32),
            pltpu.VMEM_SHARED((N_PAD, D), jnp.float32),
        ],
    )
    def k(dst_h, deg_h, idxv, rows, onesv, acc_s):
        cid = lax.axis_index("c")
        sid = lax.axis_index("s")
        wid = sid * 2 + cid

        def fillrow(r, cc):
            for j in range(D // 16):
                rows[r, pl.ds(j * 16, 16)] = jnp.zeros((16,), jnp.float32)
                onesv[r, pl.ds(j * 16, 16)] = jnp.ones((16,), jnp.float32)
            return cc

        lax.fori_loop(0, CH, fillrow, 0)

        row0 = pl.multiple_of(sid * RPT, CH)
        for off in range(0, RPT, CH):
            pltpu.sync_copy(rows, acc_s.at[pl.ds(row0 + off, CH)])

        plsc.subcore_barrier()

        def step(t, carry):
            c = t * NW + wid

            @pl.when(c < NCH)
            def _():
                base = pl.multiple_of(c * CH, CH)
                pltpu.sync_copy(dst_h.at[pl.ds(base, CH)], idxv)
                pltpu.sync_copy(onesv, acc_s.at[idxv], add=True)

            return carry

        lax.fori_loop(0, TPW, step, 0)

        plsc.subcore_barrier()

        out_row0 = pl.multiple_of(cid * N_PAD + sid * RPT, CH)
        for off in range(0, RPT, CH):
            pltpu.sync_copy(acc_s.at[pl.ds(row0 + off, CH)], rows)
            pltpu.sync_copy(rows, deg_h.at[pl.ds(out_row0 + off, CH)])

    return k(dst)


def _bn_fold(ssum, ssq, g, be):
    m = ssum / E
    v = ssq / E - m * m
    s = g / jnp.sqrt(v + 1e-5)
    t = be - m * s
    return s, t


def kernel(obj_feature, rel_feature, edges_index, params):
    p = params
    f32 = jnp.float32
    row = lambda a: jnp.reshape(a, (1, -1)).astype(f32)

    src = edges_index[0].astype(jnp.int32)
    dst = edges_index[1].astype(jnp.int32)

    g0p = p['gcn'][0]
    g1p = p['gcn'][1]
    wa0, wb0, wc0 = (g0p['nn1_w1'][:D], g0p['nn1_w1'][D:2 * D],
                     g0p['nn1_w1'][2 * D:])
    b1_0 = row(g0p['nn1_b1'])
    w2ac0 = g0p['nn1_w2'][:, :D] + g0p['nn1_w2'][:, 2 * D:]
    cmsg0 = row(g0p['nn1_b2'][:D] + g0p['nn1_b2'][2 * D:])
    w2b0 = g0p['nn1_w2'][:, D:2 * D]
    b2b0 = row(g0p['nn1_b2'][D:2 * D])
    wa1, wb1, wc1 = (g1p['nn1_w1'][:D], g1p['nn1_w1'][D:2 * D],
                     g1p['nn1_w1'][2 * D:])
    b1_1 = row(g1p['nn1_b1'])
    w2b1 = g1p['nn1_w2'][:, D:2 * D]
    b2b1 = row(g1p['nn1_b2'][D:2 * D])

    xa0, xc0 = _node_mm(obj_feature, wa0, wc0)
    gsum0 = _sc_gather_add(xa0, xc0, dst, src)

    y1, ss1, sq1 = _mm_stats(rel_feature, p['enc_w1'], row(p['enc_b1']))
    s1, t1 = _bn_fold(ss1, sq1, row(p['enc_g1']), row(p['enc_be1']))
    y2, ss2, sq2 = _affine_relu_mm_stats(y1, s1, t1, p['enc_w2'],
                                         row(p['enc_b2']))
    s2, t2 = _bn_fold(ss2, sq2, row(p['enc_g2']), row(p['enc_be2']))
    edge_feature, ewb0 = _enc_finish(y2, s2, t2, wb0, b1_0)

    msg0, ewb1 = _layer0_edge(gsum0, ewb0, w2ac0, cmsg0, w2b0, b2b0,
                              wb1, b1_1)

    degp = _sc_scatter_ones(dst)
    aggp = _sc_scatter_msg(msg0, dst)
    xa1, xc1 = _node_update(
        aggp.reshape(2, N_PAD, D), degp.reshape(2, N_PAD, D), obj_feature,
        g0p['nn2_w1'], row(g0p['nn2_b1']), g0p['nn2_w2'], row(g0p['nn2_b2']),
        wa1, wc1)

    gsum1 = _sc_gather_add(xa1, xc1, dst, src)

    e_final, cpre, ssc, sqc = _layer1_edge(gsum1, ewb1, w2b1, b2b1,
                                           p['cls_w1'], row(p['cls_b1']))
    s_c, t_c = _bn_fold(ssc, sqc, row(p['cls_g1']), row(p['cls_be1']))
    rel_cls = _cls_finish(cpre, s_c, t_c, p['cls_w2'], row(p['cls_b2']))

    return rel_cls, obj_feature, edge_feature, e_final


# speedup vs baseline: 3.1169x; 1.2474x over previous
import functools

import jax
import jax.numpy as jnp
from jax import lax
from jax.experimental import pallas as pl
from jax.experimental.pallas import tpu as pltpu
from jax.experimental.pallas import tpu_sc as plsc

N = 10000
E = 160000
D = 128
R = 26

BE = 4000
GE = E // BE
BN = 2000
GN = N // BN

CH = 128
NCH = E // CH
NW = 32
TPW = (NCH + NW - 1) // NW
NS = 16
RPT = 640
N_PAD = NS * RPT


def _full(shape):
    return pl.BlockSpec(shape, lambda i: tuple(0 for _ in shape))


def _mm_stats_body(x_ref, w_ref, b_ref, y_ref, ssum_ref, ssq_ref):
    y = jnp.dot(x_ref[...].astype(jnp.bfloat16), w_ref[...],
                preferred_element_type=jnp.float32)
    y = y + b_ref[...]
    y_ref[...] = y.astype(jnp.bfloat16)

    @pl.when(pl.program_id(0) == 0)
    def _():
        ssum_ref[...] = jnp.zeros_like(ssum_ref)
        ssq_ref[...] = jnp.zeros_like(ssq_ref)

    ssum_ref[...] += jnp.sum(y, axis=0, keepdims=True)
    ssq_ref[...] += jnp.sum(y * y, axis=0, keepdims=True)


def _mm_stats(x, w, b):
    k_in = x.shape[1]
    k_out = w.shape[1]
    return pl.pallas_call(
        _mm_stats_body,
        grid=(GE,),
        in_specs=[
            pl.BlockSpec((BE, k_in), lambda i: (i, 0)),
            _full((k_in, k_out)),
            _full((1, k_out)),
        ],
        out_specs=[
            pl.BlockSpec((BE, k_out), lambda i: (i, 0)),
            pl.BlockSpec((1, k_out), lambda i: (0, 0)),
            pl.BlockSpec((1, k_out), lambda i: (0, 0)),
        ],
        out_shape=[
            jax.ShapeDtypeStruct((E, k_out), jnp.bfloat16),
            jax.ShapeDtypeStruct((1, k_out), jnp.float32),
            jax.ShapeDtypeStruct((1, k_out), jnp.float32),
        ],
    )(x, w, b)


def _affine_relu_mm_stats_body(x_ref, s_ref, t_ref, w_ref, b_ref,
                               y_ref, ssum_ref, ssq_ref):
    h = jax.nn.relu(x_ref[...].astype(jnp.float32) * s_ref[...] + t_ref[...])
    y = jnp.dot(h.astype(jnp.bfloat16), w_ref[...],
                preferred_element_type=jnp.float32) + b_ref[...]
    y_ref[...] = y.astype(jnp.bfloat16)

    @pl.when(pl.program_id(0) == 0)
    def _():
        ssum_ref[...] = jnp.zeros_like(ssum_ref)
        ssq_ref[...] = jnp.zeros_like(ssq_ref)

    ssum_ref[...] += jnp.sum(y, axis=0, keepdims=True)
    ssq_ref[...] += jnp.sum(y * y, axis=0, keepdims=True)


def _affine_relu_mm_stats(x, s, t, w, b):
    k_in = x.shape[1]
    k_out = w.shape[1]
    return pl.pallas_call(
        _affine_relu_mm_stats_body,
        grid=(GE,),
        in_specs=[
            pl.BlockSpec((BE, k_in), lambda i: (i, 0)),
            _full((1, k_in)),
            _full((1, k_in)),
            _full((k_in, k_out)),
            _full((1, k_out)),
        ],
        out_specs=[
            pl.BlockSpec((BE, k_out), lambda i: (i, 0)),
            pl.BlockSpec((1, k_out), lambda i: (0, 0)),
            pl.BlockSpec((1, k_out), lambda i: (0, 0)),
        ],
        out_shape=[
            jax.ShapeDtypeStruct((E, k_out), jnp.bfloat16),
            jax.ShapeDtypeStruct((1, k_out), jnp.float32),
            jax.ShapeDtypeStruct((1, k_out), jnp.float32),
        ],
    )(x, s, t, w, b)


def _enc_finish_body(y_ref, s_ref, t_ref, w_ref, b_ref, ef_ref, ewb_ref):
    ef = jax.nn.relu(y_ref[...].astype(jnp.float32) * s_ref[...] + t_ref[...])
    ef_ref[...] = ef
    ewb_ref[...] = (
        jnp.dot(ef.astype(jnp.bfloat16), w_ref[...],
                preferred_element_type=jnp.float32)
        + b_ref[...]).astype(jnp.bfloat16)


def _enc_finish(y2, s, t, wb0, b1_0):
    return pl.pallas_call(
        _enc_finish_body,
        grid=(GE,),
        in_specs=[
            pl.BlockSpec((BE, D), lambda i: (i, 0)),
            _full((1, D)),
            _full((1, D)),
            _full((D, D)),
            _full((1, D)),
        ],
        out_specs=[
            pl.BlockSpec((BE, D), lambda i: (i, 0)),
            pl.BlockSpec((BE, D), lambda i: (i, 0)),
        ],
        out_shape=[
            jax.ShapeDtypeStruct((E, D), jnp.float32),
            jax.ShapeDtypeStruct((E, D), jnp.bfloat16),
        ],
    )(y2, s, t, wb0, b1_0)


def _layer0_edge_body(g_ref, ewb_ref, w2ac_ref, cmsg_ref, w2b_ref, b2b_ref,
                      wb1_ref, b11_ref, msg_ref, ewb1_ref):
    h1 = jax.nn.relu(g_ref[...] + ewb_ref[...].astype(jnp.float32))
    h1 = h1.astype(jnp.bfloat16)
    msg_ref[...] = (
        jnp.dot(h1, w2ac_ref[...], preferred_element_type=jnp.float32)
        + cmsg_ref[...])
    e1 = jax.nn.relu(
        jnp.dot(h1, w2b_ref[...], preferred_element_type=jnp.float32)
        + b2b_ref[...])
    ewb1_ref[...] = (
        jnp.dot(e1.astype(jnp.bfloat16), wb1_ref[...],
                preferred_element_type=jnp.float32)
        + b11_ref[...]).astype(jnp.bfloat16)


def _layer0_edge(g0, ewb0, w2ac, cmsg, w2b, b2b, wb1, b11):
    return pl.pallas_call(
        _layer0_edge_body,
        grid=(GE,),
        in_specs=[
            pl.BlockSpec((BE, D), lambda i: (i, 0)),
            pl.BlockSpec((BE, D), lambda i: (i, 0)),
            _full((D, D)),
            _full((1, D)),
            _full((D, D)),
            _full((1, D)),
            _full((D, D)),
            _full((1, D)),
        ],
        out_specs=[
            pl.BlockSpec((BE, D), lambda i: (i, 0)),
            pl.BlockSpec((BE, D), lambda i: (i, 0)),
        ],
        out_shape=[
            jax.ShapeDtypeStruct((E, D), jnp.float32),
            jax.ShapeDtypeStruct((E, D), jnp.bfloat16),
        ],
    )(g0, ewb0, w2ac, cmsg, w2b, b2b, wb1, b11)


def _node_mm_body(x_ref, wa_ref, wc_ref, xa_ref, xc_ref):
    x = x_ref[...].astype(jnp.bfloat16)
    xa_ref[...] = jnp.dot(x, wa_ref[...], preferred_element_type=jnp.float32)
    xc_ref[...] = jnp.dot(x, wc_ref[...], preferred_element_type=jnp.float32)


def _node_mm(x, wa, wc):
    return pl.pallas_call(
        _node_mm_body,
        grid=(GN,),
        in_specs=[
            pl.BlockSpec((BN, D), lambda i: (i, 0)),
            _full((D, D)),
            _full((D, D)),
        ],
        out_specs=[
            pl.BlockSpec((BN, D), lambda i: (i, 0)),
            pl.BlockSpec((BN, D), lambda i: (i, 0)),
        ],
        out_shape=[
            jax.ShapeDtypeStruct((N, D), jnp.float32),
            jax.ShapeDtypeStruct((N, D), jnp.float32),
        ],
    )(x, wa, wc)


def _node_update_body(aggp_ref, degp_ref, x_ref, w1_ref, b1_ref, w2_ref,
                      b2_ref, wa_ref, wc_ref, xa_ref, xc_ref):
    deg = jnp.maximum(degp_ref[0, :, 0:1] + degp_ref[1, :, 0:1], 1.0)
    agg = (aggp_ref[0] + aggp_ref[1]) / deg
    h2 = jax.nn.relu(
        jnp.dot(agg.astype(jnp.bfloat16), w1_ref[...],
                preferred_element_type=jnp.float32)
        + b1_ref[...])
    xn = x_ref[...] + (
        jnp.dot(h2.astype(jnp.bfloat16), w2_ref[...],
                preferred_element_type=jnp.float32)
        + b2_ref[...])
    xn = jax.nn.relu(xn).astype(jnp.bfloat16)
    xa_ref[...] = jnp.dot(xn, wa_ref[...], preferred_element_type=jnp.float32)
    xc_ref[...] = jnp.dot(xn, wc_ref[...], preferred_element_type=jnp.float32)


def _node_update(aggp, degp, x, w1, b1, w2, b2, wa, wc):
    return pl.pallas_call(
        _node_update_body,
        grid=(GN,),
        in_specs=[
            pl.BlockSpec((2, BN, D), lambda i: (0, i, 0)),
            pl.BlockSpec((2, BN, D), lambda i: (0, i, 0)),
            pl.BlockSpec((BN, D), lambda i: (i, 0)),
            _full((D, D)),
            _full((1, D)),
            _full((D, D)),
            _full((1, D)),
            _full((D, D)),
            _full((D, D)),
        ],
        out_specs=[
            pl.BlockSpec((BN, D), lambda i: (i, 0)),
            pl.BlockSpec((BN, D), lambda i: (i, 0)),
        ],
        out_shape=[
            jax.ShapeDtypeStruct((N, D), jnp.float32),
            jax.ShapeDtypeStruct((N, D), jnp.float32),
        ],
    )(aggp, degp, x, w1, b1, w2, b2, wa, wc)


def _layer1_edge_body(g_ref, ewb_ref, w2b_ref, b2b_ref, cw_ref, cb_ref,
                      ef_ref, cpre_ref, ssum_ref, ssq_ref):
    h1 = jax.nn.relu(g_ref[...] + ewb_ref[...].astype(jnp.float32))
    ef = (jnp.dot(h1.astype(jnp.bfloat16), w2b_ref[...],
                  preferred_element_type=jnp.float32)
          + b2b_ref[...])
    ef_ref[...] = ef
    cpre = (jnp.dot(ef.astype(jnp.bfloat16), cw_ref[...],
                    preferred_element_type=jnp.float32)
            + cb_ref[...])
    cpre_ref[...] = cpre.astype(jnp.bfloat16)

    @pl.when(pl.program_id(0) == 0)
    def _():
        ssum_ref[...] = jnp.zeros_like(ssum_ref)
        ssq_ref[...] = jnp.zeros_like(ssq_ref)

    ssum_ref[...] += jnp.sum(cpre, axis=0, keepdims=True)
    ssq_ref[...] += jnp.sum(cpre * cpre, axis=0, keepdims=True)


def _layer1_edge(g1, ewb1, w2b, b2b, cw1, cb1):
    hc = cw1.shape[1]
    return pl.pallas_call(
        _layer1_edge_body,
        grid=(GE,),
        in_specs=[
            pl.BlockSpec((BE, D), lambda i: (i, 0)),
            pl.BlockSpec((BE, D), lambda i: (i, 0)),
            _full((D, D)),
            _full((1, D)),
            _full((D, hc)),
            _full((1, hc)),
        ],
        out_specs=[
            pl.BlockSpec((BE, D), lambda i: (i, 0)),
            pl.BlockSpec((BE, hc), lambda i: (i, 0)),
            pl.BlockSpec((1, hc), lambda i: (0, 0)),
            pl.BlockSpec((1, hc), lambda i: (0, 0)),
        ],
        out_shape=[
            jax.ShapeDtypeStruct((E, D), jnp.float32),
            jax.ShapeDtypeStruct((E, hc), jnp.bfloat16),
            jax.ShapeDtypeStruct((1, hc), jnp.float32),
            jax.ShapeDtypeStruct((1, hc), jnp.float32),
        ],
    )(g1, ewb1, w2b, b2b, cw1, cb1)


def _cls_finish_body(c_ref, s_ref, t_ref, w_ref, b_ref, out_ref):
    c = jax.nn.relu(c_ref[...].astype(jnp.float32) * s_ref[...] + t_ref[...])
    logits = (jnp.dot(c.astype(jnp.bfloat16), w_ref[...],
                      preferred_element_type=jnp.float32)
              + b_ref[...])
    out_ref[...] = jax.nn.sigmoid(logits)


def _cls_finish(cpre, s, t, w, b):
    hc = cpre.shape[1]
    return pl.pallas_call(
        _cls_finish_body,
        grid=(GE,),
        in_specs=[
            pl.BlockSpec((BE, hc), lambda i: (i, 0)),
            _full((1, hc)),
            _full((1, hc)),
            _full((hc, R)),
            _full((1, R)),
        ],
        out_specs=pl.BlockSpec((BE, R), lambda i: (i, 0)),
        out_shape=jax.ShapeDtypeStruct((E, R), jnp.float32),
    )(cpre, s, t, w, b)


def _sc_gather_add(table_a, table_b, idx_a, idx_b):
    mesh = plsc.VectorSubcoreMesh(core_axis_name="c", subcore_axis_name="s")

    @functools.partial(
        pl.kernel,
        mesh=mesh,
        out_type=jax.ShapeDtypeStruct((E, D), jnp.float32),
        scratch_types=[
            pltpu.VMEM((CH,), jnp.int32),
            pltpu.VMEM((CH,), jnp.int32),
            pltpu.VMEM((CH, D), jnp.float32),
            pltpu.VMEM((CH, D), jnp.float32),
            pltpu.SemaphoreType.DMA,
            pltpu.SemaphoreType.DMA,
        ],
    )
    def k(ta_h, tb_h, ia_h, ib_h, out_h, iva, ivb, ra, rb, sa, sb):
        wid = lax.axis_index("s") * 2 + lax.axis_index("c")

        def step(t, carry):
            c = t * NW + wid

            @pl.when(c < NCH)
            def _():
                base = pl.multiple_of(c * CH, CH)
                pltpu.sync_copy(ia_h.at[pl.ds(base, CH)], iva)
                pltpu.sync_copy(ib_h.at[pl.ds(base, CH)], ivb)
                cpa = pltpu.async_copy(ta_h.at[iva], ra, sa)
                cpb = pltpu.async_copy(tb_h.at[ivb], rb, sb)
                cpa.wait()
                cpb.wait()

                def addrow(r, cc):
                    for j in range(D // 16):
                        sl = pl.ds(j * 16, 16)
                        ra[r, sl] = ra[r, sl] + rb[r, sl]
                    return cc

                lax.fori_loop(0, CH, addrow, 0)
                pltpu.sync_copy(ra, out_h.at[pl.ds(base, CH)])

            return carry

        lax.fori_loop(0, TPW, step, 0)

    return k(table_a, table_b, idx_a, idx_b)


def _sc_scatter_msg(msg, dst):
    mesh = plsc.VectorSubcoreMesh(core_axis_name="c", subcore_axis_name="s")

    @functools.partial(
        pl.kernel,
        mesh=mesh,
        out_type=jax.ShapeDtypeStruct((2 * N_PAD, D), jnp.float32),
        scratch_types=[
            pltpu.VMEM((CH,), jnp.int32),
            pltpu.VMEM((CH, D), jnp.float32),
            pltpu.VMEM_SHARED((N_PAD, D), jnp.float32),
        ],
    )
    def k(msg_h, dst_h, agg_h, idxv, rows, acc_s):
        cid = lax.axis_index("c")
        sid = lax.axis_index("s")
        wid = sid * 2 + cid

        def zrow(r, cc):
            for j in range(D // 16):
                rows[r, pl.ds(j * 16, 16)] = jnp.zeros((16,), jnp.float32)
            return cc

        lax.fori_loop(0, CH, zrow, 0)

        row0 = pl.multiple_of(sid * RPT, CH)
        for off in range(0, RPT, CH):
            pltpu.sync_copy(rows, acc_s.at[pl.ds(row0 + off, CH)])

        plsc.subcore_barrier()

        def step(t, carry):
            c = t * NW + wid

            @pl.when(c < NCH)
            def _():
                base = pl.multiple_of(c * CH, CH)
                pltpu.sync_copy(dst_h.at[pl.ds(base, CH)], idxv)
                pltpu.sync_copy(msg_h.at[pl.ds(base, CH)], rows)
                pltpu.sync_copy(rows, acc_s.at[idxv], add=True)

            return carry

        lax.fori_loop(0, TPW, step, 0)

        plsc.subcore_barrier()

        out_row0 = pl.multiple_of(cid * N_PAD + sid * RPT, CH)
        for off in range(0, RPT, CH):
            pltpu.sync_copy(acc_s.at[pl.ds(row0 + off, CH)], rows)
            pltpu.sync_copy(rows, agg_h.at[pl.ds(out_row0 + off, CH)])

    return k(msg, dst)


def _sc_scatter_ones(dst):
    mesh = plsc.VectorSubcoreMesh(core_axis_name="c", subcore_axis_name="s")

    @functools.partial(
        pl.kernel,
        mesh=mesh,
        out_type=jax.ShapeDtypeStruct((2 * N_PAD, D), jnp.float32),
        scratch_types=[
            pltpu.VMEM((CH,), jnp.int32),
            pltpu.VMEM((CH, D), jnp.float32),
            pltpu.VMEM((CH, D), jnp.float32),
            pltpu.VMEM_SHARED((N_PAD, D), jnp.float32),
        ],
    )
    def k(dst_h, deg_h, idxv, rows, onesv, acc_s):
        cid = lax.axis_index("c")
        sid = lax.axis_index("s")
        wid = sid * 2 + cid

        def fillrow(r, cc):
            for j in range(D // 16):
                rows[r, pl.ds(j * 16, 16)] = jnp.zeros((16,), jnp.float32)
                onesv[r, pl.ds(j * 16, 16)] = jnp.ones((16,), jnp.float32)
            return cc

        lax.fori_loop(0, CH, fillrow, 0)

        row0 = pl.multiple_of(sid * RPT, CH)
        for off in range(0, RPT, CH):
            pltpu.sync_copy(rows, acc_s.at[pl.ds(row0 + off, CH)])

        plsc.subcore_barrier()

        def step(t, carry):
            c = t * NW + wid

            @pl.when(c < NCH)
            def _():
                base = pl.multiple_of(c * CH, CH)
                pltpu.sync_copy(dst_h.at[pl.ds(base, CH)], idxv)
                pltpu.sync_copy(onesv, acc_s.at[idxv], add=True)

            return carry

        lax.fori_loop(0, TPW, step, 0)

        plsc.subcore_barrier()

        out_row0 = pl.multiple_of(cid * N_PAD + sid * RPT, CH)
        for off in range(0, RPT, CH):
            pltpu.sync_copy(acc_s.at[pl.ds(row0 + off, CH)], rows)
            pltpu.sync_copy(rows, deg_h.at[pl.ds(out_row0 + off, CH)])

    return k(dst)


def _bn_fold(ssum, ssq, g, be):
    m = ssum / E
    v = ssq / E - m * m
    s = g / jnp.sqrt(v + 1e-5)
    t = be - m * s
    return s, t


def kernel(obj_feature, rel_feature, edges_index, params):
    p = params
    f32 = jnp.float32
    bf = lambda a: a.astype(jnp.bfloat16)
    row = lambda a: jnp.reshape(a, (1, -1)).astype(f32)

    src = edges_index[0].astype(jnp.int32)
    dst = edges_index[1].astype(jnp.int32)

    g0p = p['gcn'][0]
    g1p = p['gcn'][1]
    wa0, wb0, wc0 = (g0p['nn1_w1'][:D], g0p['nn1_w1'][D:2 * D],
                     g0p['nn1_w1'][2 * D:])
    b1_0 = row(g0p['nn1_b1'])
    w2ac0 = g0p['nn1_w2'][:, :D] + g0p['nn1_w2'][:, 2 * D:]
    cmsg0 = row(g0p['nn1_b2'][:D] + g0p['nn1_b2'][2 * D:])
    w2b0 = g0p['nn1_w2'][:, D:2 * D]
    b2b0 = row(g0p['nn1_b2'][D:2 * D])
    wa1, wb1, wc1 = (g1p['nn1_w1'][:D], g1p['nn1_w1'][D:2 * D],
                     g1p['nn1_w1'][2 * D:])
    b1_1 = row(g1p['nn1_b1'])
    w2b1 = g1p['nn1_w2'][:, D:2 * D]
    b2b1 = row(g1p['nn1_b2'][D:2 * D])

    xa0, xc0 = _node_mm(obj_feature, bf(wa0), bf(wc0))
    gsum0 = _sc_gather_add(xa0, xc0, dst, src)

    y1, ss1, sq1 = _mm_stats(rel_feature, bf(p['enc_w1']), row(p['enc_b1']))
    s1, t1 = _bn_fold(ss1, sq1, row(p['enc_g1']), row(p['enc_be1']))
    y2, ss2, sq2 = _affine_relu_mm_stats(y1, s1, t1, bf(p['enc_w2']),
                                         row(p['enc_b2']))
    s2, t2 = _bn_fold(ss2, sq2, row(p['enc_g2']), row(p['enc_be2']))
    edge_feature, ewb0 = _enc_finish(y2, s2, t2, bf(wb0), b1_0)

    msg0, ewb1 = _layer0_edge(gsum0, ewb0, bf(w2ac0), cmsg0, bf(w2b0), b2b0,
                              bf(wb1), b1_1)

    degp = _sc_scatter_ones(dst)
    aggp = _sc_scatter_msg(msg0, dst)
    xa1, xc1 = _node_update(
        aggp.reshape(2, N_PAD, D), degp.reshape(2, N_PAD, D), obj_feature,
        bf(g0p['nn2_w1']), row(g0p['nn2_b1']), bf(g0p['nn2_w2']),
        row(g0p['nn2_b2']), bf(wa1), bf(wc1))

    gsum1 = _sc_gather_add(xa1, xc1, dst, src)

    e_final, cpre, ssc, sqc = _layer1_edge(gsum1, ewb1, bf(w2b1), b2b1,
                                           bf(p['cls_w1']), row(p['cls_b1']))
    s_c, t_c = _bn_fold(ssc, sqc, row(p['cls_g1']), row(p['cls_be1']))
    rel_cls = _cls_finish(cpre, s_c, t_c, bf(p['cls_w2']), row(p['cls_b2']))

    return rel_cls, obj_feature, edge_feature, e_final


# speedup vs baseline: 3.4236x; 1.0984x over previous
import functools

import jax
import jax.numpy as jnp
from jax import lax
from jax.experimental import pallas as pl
from jax.experimental.pallas import tpu as pltpu
from jax.experimental.pallas import tpu_sc as plsc

N = 10000
E = 160000
D = 128
R = 26

BE = 4000
GE = E // BE
BN = 2000
GN = N // BN

CH = 128
NCH = E // CH
NW = 32
TPW = (NCH + NW - 1) // NW
NS = 16
RPT = 640
N_PAD = NS * RPT


def _full(shape):
    return pl.BlockSpec(shape, lambda i: tuple(0 for _ in shape))


def _mm_stats_body(x_ref, w_ref, b_ref, y_ref, ssum_ref, ssq_ref):
    y = jnp.dot(x_ref[...].astype(jnp.bfloat16), w_ref[...],
                preferred_element_type=jnp.float32)
    y = y + b_ref[...]
    y_ref[...] = y.astype(jnp.bfloat16)

    @pl.when(pl.program_id(0) == 0)
    def _():
        ssum_ref[...] = jnp.zeros_like(ssum_ref)
        ssq_ref[...] = jnp.zeros_like(ssq_ref)

    ssum_ref[...] += jnp.sum(y, axis=0, keepdims=True)
    ssq_ref[...] += jnp.sum(y * y, axis=0, keepdims=True)


def _mm_stats(x, w, b):
    k_in = x.shape[1]
    k_out = w.shape[1]
    return pl.pallas_call(
        _mm_stats_body,
        grid=(GE,),
        in_specs=[
            pl.BlockSpec((BE, k_in), lambda i: (i, 0)),
            _full((k_in, k_out)),
            _full((1, k_out)),
        ],
        out_specs=[
            pl.BlockSpec((BE, k_out), lambda i: (i, 0)),
            pl.BlockSpec((1, k_out), lambda i: (0, 0)),
            pl.BlockSpec((1, k_out), lambda i: (0, 0)),
        ],
        out_shape=[
            jax.ShapeDtypeStruct((E, k_out), jnp.bfloat16),
            jax.ShapeDtypeStruct((1, k_out), jnp.float32),
            jax.ShapeDtypeStruct((1, k_out), jnp.float32),
        ],
    )(x, w, b)


def _affine_relu_mm_stats_body(x_ref, s_ref, t_ref, w_ref, b_ref,
                               y_ref, ssum_ref, ssq_ref):
    h = jax.nn.relu(x_ref[...].astype(jnp.float32) * s_ref[...] + t_ref[...])
    y = jnp.dot(h.astype(jnp.bfloat16), w_ref[...],
                preferred_element_type=jnp.float32) + b_ref[...]
    y_ref[...] = y.astype(jnp.bfloat16)

    @pl.when(pl.program_id(0) == 0)
    def _():
        ssum_ref[...] = jnp.zeros_like(ssum_ref)
        ssq_ref[...] = jnp.zeros_like(ssq_ref)

    ssum_ref[...] += jnp.sum(y, axis=0, keepdims=True)
    ssq_ref[...] += jnp.sum(y * y, axis=0, keepdims=True)


def _affine_relu_mm_stats(x, s, t, w, b):
    k_in = x.shape[1]
    k_out = w.shape[1]
    return pl.pallas_call(
        _affine_relu_mm_stats_body,
        grid=(GE,),
        in_specs=[
            pl.BlockSpec((BE, k_in), lambda i: (i, 0)),
            _full((1, k_in)),
            _full((1, k_in)),
            _full((k_in, k_out)),
            _full((1, k_out)),
        ],
        out_specs=[
            pl.BlockSpec((BE, k_out), lambda i: (i, 0)),
            pl.BlockSpec((1, k_out), lambda i: (0, 0)),
            pl.BlockSpec((1, k_out), lambda i: (0, 0)),
        ],
        out_shape=[
            jax.ShapeDtypeStruct((E, k_out), jnp.bfloat16),
            jax.ShapeDtypeStruct((1, k_out), jnp.float32),
            jax.ShapeDtypeStruct((1, k_out), jnp.float32),
        ],
    )(x, s, t, w, b)


def _enc_finish_body(y_ref, s_ref, t_ref, w_ref, b_ref, ef_ref, ewb_ref):
    ef = jax.nn.relu(y_ref[...].astype(jnp.float32) * s_ref[...] + t_ref[...])
    ef_ref[...] = ef
    ewb_ref[...] = (
        jnp.dot(ef.astype(jnp.bfloat16), w_ref[...],
                preferred_element_type=jnp.float32)
        + b_ref[...]).astype(jnp.bfloat16)


def _enc_finish(y2, s, t, wb0, b1_0):
    return pl.pallas_call(
        _enc_finish_body,
        grid=(GE,),
        in_specs=[
            pl.BlockSpec((BE, D), lambda i: (i, 0)),
            _full((1, D)),
            _full((1, D)),
            _full((D, D)),
            _full((1, D)),
        ],
        out_specs=[
            pl.BlockSpec((BE, D), lambda i: (i, 0)),
            pl.BlockSpec((BE, D), lambda i: (i, 0)),
        ],
        out_shape=[
            jax.ShapeDtypeStruct((E, D), jnp.float32),
            jax.ShapeDtypeStruct((E, D), jnp.bfloat16),
        ],
    )(y2, s, t, wb0, b1_0)


def _layer0_edge_body(g_ref, ewb_ref, w2ac_ref, cmsg_ref, w2b_ref, b2b_ref,
                      wb1_ref, b11_ref, msg_ref, ewb1_ref):
    h1 = jax.nn.relu(g_ref[...].astype(jnp.float32)
                     + ewb_ref[...].astype(jnp.float32))
    h1 = h1.astype(jnp.bfloat16)
    msg_ref[...] = (
        jnp.dot(h1, w2ac_ref[...], preferred_element_type=jnp.float32)
        + cmsg_ref[...])
    e1 = jax.nn.relu(
        jnp.dot(h1, w2b_ref[...], preferred_element_type=jnp.float32)
        + b2b_ref[...])
    ewb1_ref[...] = (
        jnp.dot(e1.astype(jnp.bfloat16), wb1_ref[...],
                preferred_element_type=jnp.float32)
        + b11_ref[...]).astype(jnp.bfloat16)


def _layer0_edge(g0, ewb0, w2ac, cmsg, w2b, b2b, wb1, b11):
    return pl.pallas_call(
        _layer0_edge_body,
        grid=(GE,),
        in_specs=[
            pl.BlockSpec((BE, D), lambda i: (i, 0)),
            pl.BlockSpec((BE, D), lambda i: (i, 0)),
            _full((D, D)),
            _full((1, D)),
            _full((D, D)),
            _full((1, D)),
            _full((D, D)),
            _full((1, D)),
        ],
        out_specs=[
            pl.BlockSpec((BE, D), lambda i: (i, 0)),
            pl.BlockSpec((BE, D), lambda i: (i, 0)),
        ],
        out_shape=[
            jax.ShapeDtypeStruct((E, D), jnp.float32),
            jax.ShapeDtypeStruct((E, D), jnp.bfloat16),
        ],
    )(g0, ewb0, w2ac, cmsg, w2b, b2b, wb1, b11)


def _node_mm_body(x_ref, wa_ref, wc_ref, xa_ref, xc_ref):
    x = x_ref[...].astype(jnp.bfloat16)
    xa_ref[...] = jnp.dot(x, wa_ref[...], preferred_element_type=jnp.float32)
    xc_ref[...] = jnp.dot(x, wc_ref[...], preferred_element_type=jnp.float32)


def _node_mm(x, wa, wc):
    return pl.pallas_call(
        _node_mm_body,
        grid=(GN,),
        in_specs=[
            pl.BlockSpec((BN, D), lambda i: (i, 0)),
            _full((D, D)),
            _full((D, D)),
        ],
        out_specs=[
            pl.BlockSpec((BN, D), lambda i: (i, 0)),
            pl.BlockSpec((BN, D), lambda i: (i, 0)),
        ],
        out_shape=[
            jax.ShapeDtypeStruct((N, D), jnp.float32),
            jax.ShapeDtypeStruct((N, D), jnp.float32),
        ],
    )(x, wa, wc)


def _node_update_body(aggp_ref, degp_ref, x_ref, w1_ref, b1_ref, w2_ref,
                      b2_ref, wa_ref, wc_ref, xa_ref, xc_ref):
    deg = jnp.maximum(degp_ref[0, :, 0:1] + degp_ref[1, :, 0:1], 1.0)
    agg = (aggp_ref[0] + aggp_ref[1]) / deg
    h2 = jax.nn.relu(
        jnp.dot(agg.astype(jnp.bfloat16), w1_ref[...],
                preferred_element_type=jnp.float32)
        + b1_ref[...])
    xn = x_ref[...] + (
        jnp.dot(h2.astype(jnp.bfloat16), w2_ref[...],
                preferred_element_type=jnp.float32)
        + b2_ref[...])
    xn = jax.nn.relu(xn).astype(jnp.bfloat16)
    xa_ref[...] = jnp.dot(xn, wa_ref[...], preferred_element_type=jnp.float32)
    xc_ref[...] = jnp.dot(xn, wc_ref[...], preferred_element_type=jnp.float32)


def _node_update(aggp, degp, x, w1, b1, w2, b2, wa, wc):
    return pl.pallas_call(
        _node_update_body,
        grid=(GN,),
        in_specs=[
            pl.BlockSpec((2, BN, D), lambda i: (0, i, 0)),
            pl.BlockSpec((2, BN, D), lambda i: (0, i, 0)),
            pl.BlockSpec((BN, D), lambda i: (i, 0)),
            _full((D, D)),
            _full((1, D)),
            _full((D, D)),
            _full((1, D)),
            _full((D, D)),
            _full((D, D)),
        ],
        out_specs=[
            pl.BlockSpec((BN, D), lambda i: (i, 0)),
            pl.BlockSpec((BN, D), lambda i: (i, 0)),
        ],
        out_shape=[
            jax.ShapeDtypeStruct((N, D), jnp.float32),
            jax.ShapeDtypeStruct((N, D), jnp.float32),
        ],
    )(aggp, degp, x, w1, b1, w2, b2, wa, wc)


def _layer1_edge_body(g_ref, ewb_ref, w2b_ref, b2b_ref, cw_ref, cb_ref,
                      ef_ref, cpre_ref, ssum_ref, ssq_ref):
    h1 = jax.nn.relu(g_ref[...].astype(jnp.float32)
                     + ewb_ref[...].astype(jnp.float32))
    ef = (jnp.dot(h1.astype(jnp.bfloat16), w2b_ref[...],
                  preferred_element_type=jnp.float32)
          + b2b_ref[...])
    ef_ref[...] = ef
    cpre = (jnp.dot(ef.astype(jnp.bfloat16), cw_ref[...],
                    preferred_element_type=jnp.float32)
            + cb_ref[...])
    cpre_ref[...] = cpre.astype(jnp.bfloat16)

    @pl.when(pl.program_id(0) == 0)
    def _():
        ssum_ref[...] = jnp.zeros_like(ssum_ref)
        ssq_ref[...] = jnp.zeros_like(ssq_ref)

    ssum_ref[...] += jnp.sum(cpre, axis=0, keepdims=True)
    ssq_ref[...] += jnp.sum(cpre * cpre, axis=0, keepdims=True)


def _layer1_edge(g1, ewb1, w2b, b2b, cw1, cb1):
    hc = cw1.shape[1]
    return pl.pallas_call(
        _layer1_edge_body,
        grid=(GE,),
        in_specs=[
            pl.BlockSpec((BE, D), lambda i: (i, 0)),
            pl.BlockSpec((BE, D), lambda i: (i, 0)),
            _full((D, D)),
            _full((1, D)),
            _full((D, hc)),
            _full((1, hc)),
        ],
        out_specs=[
            pl.BlockSpec((BE, D), lambda i: (i, 0)),
            pl.BlockSpec((BE, hc), lambda i: (i, 0)),
            pl.BlockSpec((1, hc), lambda i: (0, 0)),
            pl.BlockSpec((1, hc), lambda i: (0, 0)),
        ],
        out_shape=[
            jax.ShapeDtypeStruct((E, D), jnp.float32),
            jax.ShapeDtypeStruct((E, hc), jnp.bfloat16),
            jax.ShapeDtypeStruct((1, hc), jnp.float32),
            jax.ShapeDtypeStruct((1, hc), jnp.float32),
        ],
    )(g1, ewb1, w2b, b2b, cw1, cb1)


def _cls_finish_body(c_ref, s_ref, t_ref, w_ref, b_ref, out_ref):
    c = jax.nn.relu(c_ref[...].astype(jnp.float32) * s_ref[...] + t_ref[...])
    logits = (jnp.dot(c.astype(jnp.bfloat16), w_ref[...],
                      preferred_element_type=jnp.float32)
              + b_ref[...])
    out_ref[...] = jax.nn.sigmoid(logits)


def _cls_finish(cpre, s, t, w, b):
    hc = cpre.shape[1]
    return pl.pallas_call(
        _cls_finish_body,
        grid=(GE,),
        in_specs=[
            pl.BlockSpec((BE, hc), lambda i: (i, 0)),
            _full((1, hc)),
            _full((1, hc)),
            _full((hc, R)),
            _full((1, R)),
        ],
        out_specs=pl.BlockSpec((BE, R), lambda i: (i, 0)),
        out_shape=jax.ShapeDtypeStruct((E, R), jnp.float32),
    )(cpre, s, t, w, b)


def _sc_gather_add(table_a, table_b, idx_ab):
    mesh = plsc.VectorSubcoreMesh(core_axis_name="c", subcore_axis_name="s")

    @functools.partial(
        pl.kernel,
        mesh=mesh,
        out_type=jax.ShapeDtypeStruct((E, D), jnp.float32),
        scratch_types=[
            pltpu.VMEM((2, 2, CH), jnp.int32),
            pltpu.VMEM((2, CH, D), jnp.float32),
            pltpu.VMEM((2, CH, D), jnp.float32),
            pltpu.SemaphoreType.DMA,
            pltpu.SemaphoreType.DMA,
            pltpu.SemaphoreType.DMA,
            pltpu.SemaphoreType.DMA,
            pltpu.SemaphoreType.DMA,
            pltpu.SemaphoreType.DMA,
        ],
    )
    def k(ta_h, tb_h, idx_h, out_h, iv, ra, rb,
          sga0, sga1, sgb0, sgb1, swr0, swr1):
        sga = (sga0, sga1)
        sgb = (sgb0, sgb1)
        swr = (swr0, swr1)
        wid = lax.axis_index("s") * 2 + lax.axis_index("c")
        tw = (NCH - 1 - wid) // NW + 1

        def issue(t, b):
            base = pl.multiple_of((t * NW + wid) * CH, CH)
            pltpu.sync_copy(idx_h.at[:, pl.ds(base, CH)], iv.at[b])
            pltpu.async_copy(ta_h.at[iv.at[b, 0]], ra.at[b], sga[b])
            pltpu.async_copy(tb_h.at[iv.at[b, 1]], rb.at[b], sgb[b])

        for b in (0, 1):
            issue(b, b)

        def pair(g, cc):
            for b in (0, 1):
                t = 2 * g + b

                @pl.when(t < tw)
                def _():
                    base = pl.multiple_of((t * NW + wid) * CH, CH)
                    pltpu.make_async_copy(
                        out_h.at[pl.ds(0, CH)], ra.at[b], sga[b]).wait()
                    pltpu.make_async_copy(
                        out_h.at[pl.ds(0, CH)], rb.at[b], sgb[b]).wait()

                    def addrow(r, cc2):
                        for j in range(D // 16):
                            sl = pl.ds(j * 16, 16)
                            ra[b, r, sl] = ra[b, r, sl] + rb[b, r, sl]
                        return cc2

                    lax.fori_loop(0, CH, addrow, 0)
                    pltpu.async_copy(ra.at[b], out_h.at[pl.ds(base, CH)],
                                     swr[b])

                t2 = t + 2

                @pl.when(t2 < tw)
                def _():
                    pltpu.make_async_copy(
                        out_h.at[pl.ds(0, CH)], ra.at[b], swr[b]).wait()
                    issue(t2, b)
            return cc

        lax.fori_loop(0, (TPW + 1) // 2, pair, 0)

        for b in (0, 1):
            pltpu.make_async_copy(
                out_h.at[pl.ds(0, CH)], ra.at[b], swr[b]).wait()

    return k(table_a, table_b, idx_ab)


def _sc_scatter_msg(msg, dst):
    mesh = plsc.VectorSubcoreMesh(core_axis_name="c", subcore_axis_name="s")

    @functools.partial(
        pl.kernel,
        mesh=mesh,
        out_type=jax.ShapeDtypeStruct((2 * N_PAD, D), jnp.float32),
        scratch_types=[
            pltpu.VMEM((CH,), jnp.int32),
            pltpu.VMEM((CH, D), jnp.float32),
            pltpu.VMEM_SHARED((N_PAD, D), jnp.float32),
        ],
    )
    def k(msg_h, dst_h, agg_h, idxv, rows, acc_s):
        cid = lax.axis_index("c")
        sid = lax.axis_index("s")
        wid = sid * 2 + cid

        def zrow(r, cc):
            for j in range(D // 16):
                rows[r, pl.ds(j * 16, 16)] = jnp.zeros((16,), jnp.float32)
            return cc

        lax.fori_loop(0, CH, zrow, 0)

        row0 = pl.multiple_of(sid * RPT, CH)
        for off in range(0, RPT, CH):
            pltpu.sync_copy(rows, acc_s.at[pl.ds(row0 + off, CH)])

        plsc.subcore_barrier()

        def step(t, carry):
            c = t * NW + wid

            @pl.when(c < NCH)
            def _():
                base = pl.multiple_of(c * CH, CH)
                pltpu.sync_copy(dst_h.at[pl.ds(base, CH)], idxv)
                pltpu.sync_copy(msg_h.at[pl.ds(base, CH)], rows)
                pltpu.sync_copy(rows, acc_s.at[idxv], add=True)

            return carry

        lax.fori_loop(0, TPW, step, 0)

        plsc.subcore_barrier()

        out_row0 = pl.multiple_of(cid * N_PAD + sid * RPT, CH)
        for off in range(0, RPT, CH):
            pltpu.sync_copy(acc_s.at[pl.ds(row0 + off, CH)], rows)
            pltpu.sync_copy(rows, agg_h.at[pl.ds(out_row0 + off, CH)])

    return k(msg, dst)


def _sc_scatter_ones(dst):
    mesh = plsc.VectorSubcoreMesh(core_axis_name="c", subcore_axis_name="s")

    @functools.partial(
        pl.kernel,
        mesh=mesh,
        out_type=jax.ShapeDtypeStruct((2 * N_PAD, D), jnp.float32),
        scratch_types=[
            pltpu.VMEM((CH,), jnp.int32),
            pltpu.VMEM((CH, D), jnp.float32),
            pltpu.VMEM((CH, D), jnp.float32),
            pltpu.VMEM_SHARED((N_PAD, D), jnp.float32),
        ],
    )
    def k(dst_h, deg_h, idxv, rows, onesv, acc_s):
        cid = lax.axis_index("c")
        sid = lax.axis_index("s")
        wid = sid * 2 + cid

        def fillrow(r, cc):
            for j in range(D // 16):
                rows[r, pl.ds(j * 16, 16)] = jnp.zeros((16,), jnp.float32)
                onesv[r, pl.ds(j * 16, 16)] = jnp.ones((16,), jnp.float32)
            return cc

        lax.fori_loop(0, CH, fillrow, 0)

        row0 = pl.multiple_of(sid * RPT, CH)
        for off in range(0, RPT, CH):
            pltpu.sync_copy(rows, acc_s.at[pl.ds(row0 + off, CH)])

        plsc.subcore_barrier()

        def step(t, carry):
            c = t * NW + wid

            @pl.when(c < NCH)
            def _():
                base = pl.multiple_of(c * CH, CH)
                pltpu.sync_copy(dst_h.at[pl.ds(base, CH)], idxv)
                pltpu.sync_copy(onesv, acc_s.at[idxv], add=True)

            return carry

        lax.fori_loop(0, TPW, step, 0)

        plsc.subcore_barrier()

        out_row0 = pl.multiple_of(cid * N_PAD + sid * RPT, CH)
        for off in range(0, RPT, CH):
            pltpu.sync_copy(acc_s.at[pl.ds(row0 + off, CH)], rows)
            pltpu.sync_copy(rows, deg_h.at[pl.ds(out_row0 + off, CH)])

    return k(dst)


def _bn_fold(ssum, ssq, g, be):
    m = ssum / E
    v = ssq / E - m * m
    s = g / jnp.sqrt(v + 1e-5)
    t = be - m * s
    return s, t


def kernel(obj_feature, rel_feature, edges_index, params):
    p = params
    f32 = jnp.float32
    bf = lambda a: a.astype(jnp.bfloat16)
    row = lambda a: jnp.reshape(a, (1, -1)).astype(f32)

    src = edges_index[0].astype(jnp.int32)
    dst = edges_index[1].astype(jnp.int32)
    idx_ds = jnp.stack([dst, src])

    g0p = p['gcn'][0]
    g1p = p['gcn'][1]
    wa0, wb0, wc0 = (g0p['nn1_w1'][:D], g0p['nn1_w1'][D:2 * D],
                     g0p['nn1_w1'][2 * D:])
    b1_0 = row(g0p['nn1_b1'])
    w2ac0 = g0p['nn1_w2'][:, :D] + g0p['nn1_w2'][:, 2 * D:]
    cmsg0 = row(g0p['nn1_b2'][:D] + g0p['nn1_b2'][2 * D:])
    w2b0 = g0p['nn1_w2'][:, D:2 * D]
    b2b0 = row(g0p['nn1_b2'][D:2 * D])
    wa1, wb1, wc1 = (g1p['nn1_w1'][:D], g1p['nn1_w1'][D:2 * D],
                     g1p['nn1_w1'][2 * D:])
    b1_1 = row(g1p['nn1_b1'])
    w2b1 = g1p['nn1_w2'][:, D:2 * D]
    b2b1 = row(g1p['nn1_b2'][D:2 * D])

    xa0, xc0 = _node_mm(obj_feature, bf(wa0), bf(wc0))
    gsum0 = _sc_gather_add(xa0, xc0, idx_ds)

    y1, ss1, sq1 = _mm_stats(rel_feature, bf(p['enc_w1']), row(p['enc_b1']))
    s1, t1 = _bn_fold(ss1, sq1, row(p['enc_g1']), row(p['enc_be1']))
    y2, ss2, sq2 = _affine_relu_mm_stats(y1, s1, t1, bf(p['enc_w2']),
                                         row(p['enc_b2']))
    s2, t2 = _bn_fold(ss2, sq2, row(p['enc_g2']), row(p['enc_be2']))
    edge_feature, ewb0 = _enc_finish(y2, s2, t2, bf(wb0), b1_0)

    msg0, ewb1 = _layer0_edge(gsum0, ewb0, bf(w2ac0), cmsg0, bf(w2b0), b2b0,
                              bf(wb1), b1_1)

    degp = _sc_scatter_ones(dst)
    aggp = _sc_scatter_msg(msg0, dst)
    xa1, xc1 = _node_update(
        aggp.reshape(2, N_PAD, D), degp.reshape(2, N_PAD, D), obj_feature,
        bf(g0p['nn2_w1']), row(g0p['nn2_b1']), bf(g0p['nn2_w2']),
        row(g0p['nn2_b2']), bf(wa1), bf(wc1))

    gsum1 = _sc_gather_add(xa1, xc1, idx_ds)

    e_final, cpre, ssc, sqc = _layer1_edge(gsum1, ewb1, bf(w2b1), b2b1,
                                           bf(p['cls_w1']), row(p['cls_b1']))
    s_c, t_c = _bn_fold(ssc, sqc, row(p['cls_g1']), row(p['cls_be1']))
    rel_cls = _cls_finish(cpre, s_c, t_c, bf(p['cls_w2']), row(p['cls_b2']))

    return rel_cls, obj_feature, edge_feature, e_final


# speedup vs baseline: 3.8524x; 1.1252x over previous
import functools

import jax
import jax.numpy as jnp
from jax import lax
from jax.experimental import pallas as pl
from jax.experimental.pallas import tpu as pltpu
from jax.experimental.pallas import tpu_sc as plsc

N = 10000
E = 160000
D = 128
R = 26

BE = 4000
GE = E // BE
BN = 2000
GN = N // BN

CH = 128
NCH = E // CH
NW = 32
TPW = (NCH + NW - 1) // NW
NS = 16
RPT = 640
N_PAD = NS * RPT


def _full(shape):
    return pl.BlockSpec(shape, lambda i: tuple(0 for _ in shape))


def _mm_stats_body(x_ref, w_ref, b_ref, y_ref, ssum_ref, ssq_ref):
    y = jnp.dot(x_ref[...].astype(jnp.bfloat16), w_ref[...],
                preferred_element_type=jnp.float32)
    y = y + b_ref[...]
    y_ref[...] = y.astype(jnp.bfloat16)

    @pl.when(pl.program_id(0) == 0)
    def _():
        ssum_ref[...] = jnp.zeros_like(ssum_ref)
        ssq_ref[...] = jnp.zeros_like(ssq_ref)

    ssum_ref[...] += jnp.sum(y, axis=0, keepdims=True)
    ssq_ref[...] += jnp.sum(y * y, axis=0, keepdims=True)


def _mm_stats(x, w, b):
    k_in = x.shape[1]
    k_out = w.shape[1]
    return pl.pallas_call(
        _mm_stats_body,
        grid=(GE,),
        in_specs=[
            pl.BlockSpec((BE, k_in), lambda i: (i, 0)),
            _full((k_in, k_out)),
            _full((1, k_out)),
        ],
        out_specs=[
            pl.BlockSpec((BE, k_out), lambda i: (i, 0)),
            pl.BlockSpec((1, k_out), lambda i: (0, 0)),
            pl.BlockSpec((1, k_out), lambda i: (0, 0)),
        ],
        out_shape=[
            jax.ShapeDtypeStruct((E, k_out), jnp.bfloat16),
            jax.ShapeDtypeStruct((1, k_out), jnp.float32),
            jax.ShapeDtypeStruct((1, k_out), jnp.float32),
        ],
    )(x, w, b)


def _affine_relu_mm_stats_body(x_ref, s_ref, t_ref, w_ref, b_ref,
                               y_ref, ssum_ref, ssq_ref):
    h = jax.nn.relu(x_ref[...].astype(jnp.float32) * s_ref[...] + t_ref[...])
    y = jnp.dot(h.astype(jnp.bfloat16), w_ref[...],
                preferred_element_type=jnp.float32) + b_ref[...]
    y_ref[...] = y.astype(jnp.bfloat16)

    @pl.when(pl.program_id(0) == 0)
    def _():
        ssum_ref[...] = jnp.zeros_like(ssum_ref)
        ssq_ref[...] = jnp.zeros_like(ssq_ref)

    ssum_ref[...] += jnp.sum(y, axis=0, keepdims=True)
    ssq_ref[...] += jnp.sum(y * y, axis=0, keepdims=True)


def _affine_relu_mm_stats(x, s, t, w, b):
    k_in = x.shape[1]
    k_out = w.shape[1]
    return pl.pallas_call(
        _affine_relu_mm_stats_body,
        grid=(GE,),
        in_specs=[
            pl.BlockSpec((BE, k_in), lambda i: (i, 0)),
            _full((1, k_in)),
            _full((1, k_in)),
            _full((k_in, k_out)),
            _full((1, k_out)),
        ],
        out_specs=[
            pl.BlockSpec((BE, k_out), lambda i: (i, 0)),
            pl.BlockSpec((1, k_out), lambda i: (0, 0)),
            pl.BlockSpec((1, k_out), lambda i: (0, 0)),
        ],
        out_shape=[
            jax.ShapeDtypeStruct((E, k_out), jnp.bfloat16),
            jax.ShapeDtypeStruct((1, k_out), jnp.float32),
            jax.ShapeDtypeStruct((1, k_out), jnp.float32),
        ],
    )(x, s, t, w, b)


def _enc_finish_body(y_ref, s_ref, t_ref, w_ref, b_ref, ef_ref, ewb_ref):
    ef = jax.nn.relu(y_ref[...].astype(jnp.float32) * s_ref[...] + t_ref[...])
    ef_ref[...] = ef
    ewb_ref[...] = (
        jnp.dot(ef.astype(jnp.bfloat16), w_ref[...],
                preferred_element_type=jnp.float32)
        + b_ref[...]).astype(jnp.bfloat16)


def _enc_finish(y2, s, t, wb0, b1_0):
    return pl.pallas_call(
        _enc_finish_body,
        grid=(GE,),
        in_specs=[
            pl.BlockSpec((BE, D), lambda i: (i, 0)),
            _full((1, D)),
            _full((1, D)),
            _full((D, D)),
            _full((1, D)),
        ],
        out_specs=[
            pl.BlockSpec((BE, D), lambda i: (i, 0)),
            pl.BlockSpec((BE, D), lambda i: (i, 0)),
        ],
        out_shape=[
            jax.ShapeDtypeStruct((E, D), jnp.float32),
            jax.ShapeDtypeStruct((E, D), jnp.bfloat16),
        ],
    )(y2, s, t, wb0, b1_0)


def _layer0_edge_body(g_ref, ewb_ref, w2ac_ref, cmsg_ref, w2b_ref, b2b_ref,
                      wb1_ref, b11_ref, msg_ref, ewb1_ref):
    h1 = jax.nn.relu(g_ref[...].astype(jnp.float32)
                     + ewb_ref[...].astype(jnp.float32))
    h1 = h1.astype(jnp.bfloat16)
    msg_ref[...] = (
        jnp.dot(h1, w2ac_ref[...], preferred_element_type=jnp.float32)
        + cmsg_ref[...])
    e1 = jax.nn.relu(
        jnp.dot(h1, w2b_ref[...], preferred_element_type=jnp.float32)
        + b2b_ref[...])
    ewb1_ref[...] = (
        jnp.dot(e1.astype(jnp.bfloat16), wb1_ref[...],
                preferred_element_type=jnp.float32)
        + b11_ref[...]).astype(jnp.bfloat16)


def _layer0_edge(g0, ewb0, w2ac, cmsg, w2b, b2b, wb1, b11):
    return pl.pallas_call(
        _layer0_edge_body,
        grid=(GE,),
        in_specs=[
            pl.BlockSpec((BE, D), lambda i: (i, 0)),
            pl.BlockSpec((BE, D), lambda i: (i, 0)),
            _full((D, D)),
            _full((1, D)),
            _full((D, D)),
            _full((1, D)),
            _full((D, D)),
            _full((1, D)),
        ],
        out_specs=[
            pl.BlockSpec((BE, D), lambda i: (i, 0)),
            pl.BlockSpec((BE, D), lambda i: (i, 0)),
        ],
        out_shape=[
            jax.ShapeDtypeStruct((E, D), jnp.float32),
            jax.ShapeDtypeStruct((E, D), jnp.bfloat16),
        ],
    )(g0, ewb0, w2ac, cmsg, w2b, b2b, wb1, b11)


def _node_mm_body(x_ref, wa_ref, wc_ref, xa_ref, xc_ref):
    x = x_ref[...].astype(jnp.bfloat16)
    xa_ref[...] = jnp.dot(x, wa_ref[...], preferred_element_type=jnp.float32)
    xc_ref[...] = jnp.dot(x, wc_ref[...], preferred_element_type=jnp.float32)


def _node_mm(x, wa, wc):
    return pl.pallas_call(
        _node_mm_body,
        grid=(GN,),
        in_specs=[
            pl.BlockSpec((BN, D), lambda i: (i, 0)),
            _full((D, D)),
            _full((D, D)),
        ],
        out_specs=[
            pl.BlockSpec((BN, D), lambda i: (i, 0)),
            pl.BlockSpec((BN, D), lambda i: (i, 0)),
        ],
        out_shape=[
            jax.ShapeDtypeStruct((N, D), jnp.float32),
            jax.ShapeDtypeStruct((N, D), jnp.float32),
        ],
    )(x, wa, wc)


def _node_update_body(aggp_ref, degp_ref, x_ref, w1_ref, b1_ref, w2_ref,
                      b2_ref, wa_ref, wc_ref, xa_ref, xc_ref):
    deg = jnp.maximum(degp_ref[0, :, 0:1] + degp_ref[1, :, 0:1], 1.0)
    agg = (aggp_ref[0] + aggp_ref[1]) / deg
    h2 = jax.nn.relu(
        jnp.dot(agg.astype(jnp.bfloat16), w1_ref[...],
                preferred_element_type=jnp.float32)
        + b1_ref[...])
    xn = x_ref[...] + (
        jnp.dot(h2.astype(jnp.bfloat16), w2_ref[...],
                preferred_element_type=jnp.float32)
        + b2_ref[...])
    xn = jax.nn.relu(xn).astype(jnp.bfloat16)
    xa_ref[...] = jnp.dot(xn, wa_ref[...], preferred_element_type=jnp.float32)
    xc_ref[...] = jnp.dot(xn, wc_ref[...], preferred_element_type=jnp.float32)


def _node_update(aggp, degp, x, w1, b1, w2, b2, wa, wc):
    return pl.pallas_call(
        _node_update_body,
        grid=(GN,),
        in_specs=[
            pl.BlockSpec((2, BN, D), lambda i: (0, i, 0)),
            pl.BlockSpec((2, BN, D), lambda i: (0, i, 0)),
            pl.BlockSpec((BN, D), lambda i: (i, 0)),
            _full((D, D)),
            _full((1, D)),
            _full((D, D)),
            _full((1, D)),
            _full((D, D)),
            _full((D, D)),
        ],
        out_specs=[
            pl.BlockSpec((BN, D), lambda i: (i, 0)),
            pl.BlockSpec((BN, D), lambda i: (i, 0)),
        ],
        out_shape=[
            jax.ShapeDtypeStruct((N, D), jnp.float32),
            jax.ShapeDtypeStruct((N, D), jnp.float32),
        ],
    )(aggp, degp, x, w1, b1, w2, b2, wa, wc)


def _layer1_edge_body(g_ref, ewb_ref, w2b_ref, b2b_ref, cw_ref, cb_ref,
                      ef_ref, cpre_ref, ssum_ref, ssq_ref):
    h1 = jax.nn.relu(g_ref[...].astype(jnp.float32)
                     + ewb_ref[...].astype(jnp.float32))
    ef = (jnp.dot(h1.astype(jnp.bfloat16), w2b_ref[...],
                  preferred_element_type=jnp.float32)
          + b2b_ref[...])
    ef_ref[...] = ef
    cpre = (jnp.dot(ef.astype(jnp.bfloat16), cw_ref[...],
                    preferred_element_type=jnp.float32)
            + cb_ref[...])
    cpre_ref[...] = cpre.astype(jnp.bfloat16)

    @pl.when(pl.program_id(0) == 0)
    def _():
        ssum_ref[...] = jnp.zeros_like(ssum_ref)
        ssq_ref[...] = jnp.zeros_like(ssq_ref)

    ssum_ref[...] += jnp.sum(cpre, axis=0, keepdims=True)
    ssq_ref[...] += jnp.sum(cpre * cpre, axis=0, keepdims=True)


def _layer1_edge(g1, ewb1, w2b, b2b, cw1, cb1):
    hc = cw1.shape[1]
    return pl.pallas_call(
        _layer1_edge_body,
        grid=(GE,),
        in_specs=[
            pl.BlockSpec((BE, D), lambda i: (i, 0)),
            pl.BlockSpec((BE, D), lambda i: (i, 0)),
            _full((D, D)),
            _full((1, D)),
            _full((D, hc)),
            _full((1, hc)),
        ],
        out_specs=[
            pl.BlockSpec((BE, D), lambda i: (i, 0)),
            pl.BlockSpec((BE, hc), lambda i: (i, 0)),
            pl.BlockSpec((1, hc), lambda i: (0, 0)),
            pl.BlockSpec((1, hc), lambda i: (0, 0)),
        ],
        out_shape=[
            jax.ShapeDtypeStruct((E, D), jnp.float32),
            jax.ShapeDtypeStruct((E, hc), jnp.bfloat16),
            jax.ShapeDtypeStruct((1, hc), jnp.float32),
            jax.ShapeDtypeStruct((1, hc), jnp.float32),
        ],
    )(g1, ewb1, w2b, b2b, cw1, cb1)


def _cls_finish_body(c_ref, s_ref, t_ref, w_ref, b_ref, out_ref):
    c = jax.nn.relu(c_ref[...].astype(jnp.float32) * s_ref[...] + t_ref[...])
    logits = lax.dot_general(
        w_ref[...], c.astype(jnp.bfloat16), (((1,), (1,)), ((), ())),
        preferred_element_type=jnp.float32) + b_ref[...]
    out_ref[...] = jax.nn.sigmoid(logits)


RP = 32
BEC = 3200
GEC = E // BEC


def _cls_finish(cpre, s, t, wT, bT):
    hc = cpre.shape[1]
    return pl.pallas_call(
        _cls_finish_body,
        grid=(GEC,),
        in_specs=[
            pl.BlockSpec((BEC, hc), lambda i: (i, 0)),
            _full((1, hc)),
            _full((1, hc)),
            _full((RP, hc)),
            _full((RP, 1)),
        ],
        out_specs=pl.BlockSpec((RP, BEC), lambda i: (0, i)),
        out_shape=jax.ShapeDtypeStruct((RP, E), jnp.float32),
    )(cpre, s, t, wT, bT)


def _sc_gather_add(table_a, table_b, idx_ab):
    mesh = plsc.VectorSubcoreMesh(core_axis_name="c", subcore_axis_name="s")

    @functools.partial(
        pl.kernel,
        mesh=mesh,
        out_type=jax.ShapeDtypeStruct((E, D), jnp.float32),
        scratch_types=[
            pltpu.VMEM((2, 2, CH), jnp.int32),
            pltpu.VMEM((2, CH, D), jnp.float32),
            pltpu.VMEM((2, CH, D), jnp.float32),
            pltpu.SemaphoreType.DMA,
            pltpu.SemaphoreType.DMA,
            pltpu.SemaphoreType.DMA,
            pltpu.SemaphoreType.DMA,
            pltpu.SemaphoreType.DMA,
            pltpu.SemaphoreType.DMA,
        ],
    )
    def k(ta_h, tb_h, idx_h, out_h, iv, ra, rb,
          sga0, sga1, sgb0, sgb1, swr0, swr1):
        sga = (sga0, sga1)
        sgb = (sgb0, sgb1)
        swr = (swr0, swr1)
        wid = lax.axis_index("s") * 2 + lax.axis_index("c")
        tw = (NCH - 1 - wid) // NW + 1

        def issue(t, b):
            base = pl.multiple_of((t * NW + wid) * CH, CH)
            pltpu.sync_copy(idx_h.at[:, pl.ds(base, CH)], iv.at[b])
            pltpu.async_copy(ta_h.at[iv.at[b, 0]], ra.at[b], sga[b])
            pltpu.async_copy(tb_h.at[iv.at[b, 1]], rb.at[b], sgb[b])

        for b in (0, 1):
            issue(b, b)

        def pair(g, cc):
            for b in (0, 1):
                t = 2 * g + b

                @pl.when(t < tw)
                def _():
                    base = pl.multiple_of((t * NW + wid) * CH, CH)
                    pltpu.make_async_copy(
                        out_h.at[pl.ds(0, CH)], ra.at[b], sga[b]).wait()
                    pltpu.make_async_copy(
                        out_h.at[pl.ds(0, CH)], rb.at[b], sgb[b]).wait()

                    def addrow(r, cc2):
                        for j in range(D // 16):
                            sl = pl.ds(j * 16, 16)
                            ra[b, r, sl] = ra[b, r, sl] + rb[b, r, sl]
                        return cc2

                    lax.fori_loop(0, CH, addrow, 0)
                    pltpu.async_copy(ra.at[b], out_h.at[pl.ds(base, CH)],
                                     swr[b])

                t2 = t + 2

                @pl.when(t2 < tw)
                def _():
                    pltpu.make_async_copy(
                        out_h.at[pl.ds(0, CH)], ra.at[b], swr[b]).wait()
                    issue(t2, b)
            return cc

        lax.fori_loop(0, (TPW + 1) // 2, pair, 0)

        for b in (0, 1):
            pltpu.make_async_copy(
                out_h.at[pl.ds(0, CH)], ra.at[b], swr[b]).wait()

    return k(table_a, table_b, idx_ab)


def _sc_scatter_msg(msg, dst):
    mesh = plsc.VectorSubcoreMesh(core_axis_name="c", subcore_axis_name="s")

    @functools.partial(
        pl.kernel,
        mesh=mesh,
        out_type=jax.ShapeDtypeStruct((2 * N_PAD, D), jnp.float32),
        scratch_types=[
            pltpu.VMEM((2, CH), jnp.int32),
            pltpu.VMEM((2, CH, D), jnp.float32),
            pltpu.SemaphoreType.DMA,
            pltpu.SemaphoreType.DMA,
            pltpu.VMEM_SHARED((N_PAD, D), jnp.float32),
        ],
    )
    def k(msg_h, dst_h, agg_h, idxv, rows, sm0, sm1, acc_s):
        sm = (sm0, sm1)
        cid = lax.axis_index("c")
        sid = lax.axis_index("s")
        wid = sid * 2 + cid
        tw = (NCH - 1 - wid) // NW + 1

        def zrow(r, cc):
            for j in range(D // 16):
                rows[0, r, pl.ds(j * 16, 16)] = jnp.zeros((16,), jnp.float32)
            return cc

        lax.fori_loop(0, CH, zrow, 0)

        row0 = pl.multiple_of(sid * RPT, CH)
        for off in range(0, RPT, CH):
            pltpu.sync_copy(rows.at[0], acc_s.at[pl.ds(row0 + off, CH)])

        plsc.subcore_barrier()

        def fetch(t, b):
            base = pl.multiple_of((t * NW + wid) * CH, CH)
            pltpu.async_copy(msg_h.at[pl.ds(base, CH)], rows.at[b], sm[b])

        for b in (0, 1):
            fetch(b, b)

        def pair(g, cc):
            for b in (0, 1):
                t = 2 * g + b

                @pl.when(t < tw)
                def _():
                    base = pl.multiple_of((t * NW + wid) * CH, CH)
                    pltpu.sync_copy(dst_h.at[pl.ds(base, CH)], idxv.at[b])
                    pltpu.make_async_copy(
                        msg_h.at[pl.ds(0, CH)], rows.at[b], sm[b]).wait()
                    pltpu.sync_copy(rows.at[b], acc_s.at[idxv.at[b]],
                                    add=True)

                t2 = t + 2

                @pl.when(t2 < tw)
                def _():
                    fetch(t2, b)
            return cc

        lax.fori_loop(0, (TPW + 1) // 2, pair, 0)

        plsc.subcore_barrier()

        out_row0 = pl.multiple_of(cid * N_PAD + sid * RPT, CH)
        for off in range(0, RPT, CH):
            pltpu.sync_copy(acc_s.at[pl.ds(row0 + off, CH)], rows.at[0])
            pltpu.sync_copy(rows.at[0], agg_h.at[pl.ds(out_row0 + off, CH)])

    return k(msg, dst)


def _sc_scatter_ones(dst):
    mesh = plsc.VectorSubcoreMesh(core_axis_name="c", subcore_axis_name="s")

    @functools.partial(
        pl.kernel,
        mesh=mesh,
        out_type=jax.ShapeDtypeStruct((2 * N_PAD, D), jnp.float32),
        scratch_types=[
            pltpu.VMEM((CH,), jnp.int32),
            pltpu.VMEM((CH, D), jnp.float32),
            pltpu.VMEM((CH, D), jnp.float32),
            pltpu.VMEM_SHARED((N_PAD, D), jnp.float32),
        ],
    )
    def k(dst_h, deg_h, idxv, rows, onesv, acc_s):
        cid = lax.axis_index("c")
        sid = lax.axis_index("s")
        wid = sid * 2 + cid

        def fillrow(r, cc):
            for j in range(D // 16):
                rows[r, pl.ds(j * 16, 16)] = jnp.zeros((16,), jnp.float32)
                onesv[r, pl.ds(j * 16, 16)] = jnp.ones((16,), jnp.float32)
            return cc

        lax.fori_loop(0, CH, fillrow, 0)

        row0 = pl.multiple_of(sid * RPT, CH)
        for off in range(0, RPT, CH):
            pltpu.sync_copy(rows, acc_s.at[pl.ds(row0 + off, CH)])

        plsc.subcore_barrier()

        def step(t, carry):
            c = t * NW + wid

            @pl.when(c < NCH)
            def _():
                base = pl.multiple_of(c * CH, CH)
                pltpu.sync_copy(dst_h.at[pl.ds(base, CH)], idxv)
                pltpu.sync_copy(onesv, acc_s.at[idxv], add=True)

            return carry

        lax.fori_loop(0, TPW, step, 0)

        plsc.subcore_barrier()

        out_row0 = pl.multiple_of(cid * N_PAD + sid * RPT, CH)
        for off in range(0, RPT, CH):
            pltpu.sync_copy(acc_s.at[pl.ds(row0 + off, CH)], rows)
            pltpu.sync_copy(rows, deg_h.at[pl.ds(out_row0 + off, CH)])

    return k(dst)


def _bn_fold(ssum, ssq, g, be):
    m = ssum / E
    v = ssq / E - m * m
    s = g / jnp.sqrt(v + 1e-5)
    t = be - m * s
    return s, t


def kernel(obj_feature, rel_feature, edges_index, params):
    p = params
    f32 = jnp.float32
    bf = lambda a: a.astype(jnp.bfloat16)
    row = lambda a: jnp.reshape(a, (1, -1)).astype(f32)

    src = edges_index[0].astype(jnp.int32)
    dst = edges_index[1].astype(jnp.int32)
    idx_ds = jnp.stack([dst, src])

    g0p = p['gcn'][0]
    g1p = p['gcn'][1]
    wa0, wb0, wc0 = (g0p['nn1_w1'][:D], g0p['nn1_w1'][D:2 * D],
                     g0p['nn1_w1'][2 * D:])
    b1_0 = row(g0p['nn1_b1'])
    w2ac0 = g0p['nn1_w2'][:, :D] + g0p['nn1_w2'][:, 2 * D:]
    cmsg0 = row(g0p['nn1_b2'][:D] + g0p['nn1_b2'][2 * D:])
    w2b0 = g0p['nn1_w2'][:, D:2 * D]
    b2b0 = row(g0p['nn1_b2'][D:2 * D])
    wa1, wb1, wc1 = (g1p['nn1_w1'][:D], g1p['nn1_w1'][D:2 * D],
                     g1p['nn1_w1'][2 * D:])
    b1_1 = row(g1p['nn1_b1'])
    w2b1 = g1p['nn1_w2'][:, D:2 * D]
    b2b1 = row(g1p['nn1_b2'][D:2 * D])

    xa0, xc0 = _node_mm(obj_feature, bf(wa0), bf(wc0))
    gsum0 = _sc_gather_add(xa0, xc0, idx_ds)

    y1, ss1, sq1 = _mm_stats(rel_feature, bf(p['enc_w1']), row(p['enc_b1']))
    s1, t1 = _bn_fold(ss1, sq1, row(p['enc_g1']), row(p['enc_be1']))
    y2, ss2, sq2 = _affine_relu_mm_stats(y1, s1, t1, bf(p['enc_w2']),
                                         row(p['enc_b2']))
    s2, t2 = _bn_fold(ss2, sq2, row(p['enc_g2']), row(p['enc_be2']))
    edge_feature, ewb0 = _enc_finish(y2, s2, t2, bf(wb0), b1_0)

    msg0, ewb1 = _layer0_edge(gsum0, ewb0, bf(w2ac0), cmsg0, bf(w2b0), b2b0,
                              bf(wb1), b1_1)

    degp = _sc_scatter_ones(dst)
    aggp = _sc_scatter_msg(msg0, dst)
    xa1, xc1 = _node_update(
        aggp.reshape(2, N_PAD, D), degp.reshape(2, N_PAD, D), obj_feature,
        bf(g0p['nn2_w1']), row(g0p['nn2_b1']), bf(g0p['nn2_w2']),
        row(g0p['nn2_b2']), bf(wa1), bf(wc1))

    gsum1 = _sc_gather_add(xa1, xc1, idx_ds)

    e_final, cpre, ssc, sqc = _layer1_edge(gsum1, ewb1, bf(w2b1), b2b1,
                                           bf(p['cls_w1']), row(p['cls_b1']))
    s_c, t_c = _bn_fold(ssc, sqc, row(p['cls_g1']), row(p['cls_be1']))
    w2t = jnp.zeros((RP, p['cls_w2'].shape[0]), jnp.bfloat16)
    w2t = w2t.at[:R].set(bf(p['cls_w2'].T))
    b2t = jnp.zeros((RP, 1), f32).at[:R].set(
        jnp.reshape(p['cls_b2'], (-1, 1)).astype(f32))
    rel_cls = _cls_finish(cpre, s_c, t_c, w2t, b2t)[:R].T

    return rel_cls, obj_feature, edge_feature, e_final


# speedup vs baseline: 4.0609x; 1.0541x over previous
import functools

import jax
import jax.numpy as jnp
from jax import lax
from jax.experimental import pallas as pl
from jax.experimental.pallas import tpu as pltpu
from jax.experimental.pallas import tpu_sc as plsc

N = 10000
E = 160000
D = 128
R = 26

BE = 8000
GE = E // BE
BN = 2000
GN = N // BN

CH = 128
NCH = E // CH
NW = 32
TPW = (NCH + NW - 1) // NW
NS = 16
RPT = 640
N_PAD = NS * RPT


def _full(shape):
    return pl.BlockSpec(shape, lambda i: tuple(0 for _ in shape))


def _mm_stats_body(x_ref, w_ref, b_ref, y_ref, ssum_ref, ssq_ref):
    y = jnp.dot(x_ref[...].astype(jnp.bfloat16), w_ref[...],
                preferred_element_type=jnp.float32)
    y = y + b_ref[...]
    y_ref[...] = y.astype(jnp.bfloat16)

    @pl.when(pl.program_id(0) == 0)
    def _():
        ssum_ref[...] = jnp.zeros_like(ssum_ref)
        ssq_ref[...] = jnp.zeros_like(ssq_ref)

    ssum_ref[...] += jnp.sum(y, axis=0, keepdims=True)
    ssq_ref[...] += jnp.sum(y * y, axis=0, keepdims=True)


def _mm_stats(x, w, b):
    k_in = x.shape[1]
    k_out = w.shape[1]
    return pl.pallas_call(
        _mm_stats_body,
        grid=(GE,),
        in_specs=[
            pl.BlockSpec((BE, k_in), lambda i: (i, 0)),
            _full((k_in, k_out)),
            _full((1, k_out)),
        ],
        out_specs=[
            pl.BlockSpec((BE, k_out), lambda i: (i, 0)),
            pl.BlockSpec((1, k_out), lambda i: (0, 0)),
            pl.BlockSpec((1, k_out), lambda i: (0, 0)),
        ],
        out_shape=[
            jax.ShapeDtypeStruct((E, k_out), jnp.bfloat16),
            jax.ShapeDtypeStruct((1, k_out), jnp.float32),
            jax.ShapeDtypeStruct((1, k_out), jnp.float32),
        ],
    )(x, w, b)


def _affine_relu_mm_stats_body(x_ref, s_ref, t_ref, w_ref, b_ref,
                               y_ref, ssum_ref, ssq_ref):
    h = jax.nn.relu(x_ref[...].astype(jnp.float32) * s_ref[...] + t_ref[...])
    y = jnp.dot(h.astype(jnp.bfloat16), w_ref[...],
                preferred_element_type=jnp.float32) + b_ref[...]
    y_ref[...] = y.astype(jnp.bfloat16)

    @pl.when(pl.program_id(0) == 0)
    def _():
        ssum_ref[...] = jnp.zeros_like(ssum_ref)
        ssq_ref[...] = jnp.zeros_like(ssq_ref)

    ssum_ref[...] += jnp.sum(y, axis=0, keepdims=True)
    ssq_ref[...] += jnp.sum(y * y, axis=0, keepdims=True)


def _affine_relu_mm_stats(x, s, t, w, b):
    k_in = x.shape[1]
    k_out = w.shape[1]
    return pl.pallas_call(
        _affine_relu_mm_stats_body,
        grid=(GE,),
        in_specs=[
            pl.BlockSpec((BE, k_in), lambda i: (i, 0)),
            _full((1, k_in)),
            _full((1, k_in)),
            _full((k_in, k_out)),
            _full((1, k_out)),
        ],
        out_specs=[
            pl.BlockSpec((BE, k_out), lambda i: (i, 0)),
            pl.BlockSpec((1, k_out), lambda i: (0, 0)),
            pl.BlockSpec((1, k_out), lambda i: (0, 0)),
        ],
        out_shape=[
            jax.ShapeDtypeStruct((E, k_out), jnp.bfloat16),
            jax.ShapeDtypeStruct((1, k_out), jnp.float32),
            jax.ShapeDtypeStruct((1, k_out), jnp.float32),
        ],
    )(x, s, t, w, b)


def _enc_finish_body(y_ref, s_ref, t_ref, w_ref, b_ref, ef_ref, ewb_ref):
    ef = jax.nn.relu(y_ref[...].astype(jnp.float32) * s_ref[...] + t_ref[...])
    ef_ref[...] = ef
    ewb_ref[...] = (
        jnp.dot(ef.astype(jnp.bfloat16), w_ref[...],
                preferred_element_type=jnp.float32)
        + b_ref[...]).astype(jnp.bfloat16)


def _enc_finish(y2, s, t, wb0, b1_0):
    return pl.pallas_call(
        _enc_finish_body,
        grid=(GE,),
        in_specs=[
            pl.BlockSpec((BE, D), lambda i: (i, 0)),
            _full((1, D)),
            _full((1, D)),
            _full((D, D)),
            _full((1, D)),
        ],
        out_specs=[
            pl.BlockSpec((BE, D), lambda i: (i, 0)),
            pl.BlockSpec((BE, D), lambda i: (i, 0)),
        ],
        out_shape=[
            jax.ShapeDtypeStruct((E, D), jnp.float32),
            jax.ShapeDtypeStruct((E, D), jnp.bfloat16),
        ],
    )(y2, s, t, wb0, b1_0)


def _layer0_edge_body(g_ref, ewb_ref, w2ac_ref, cmsg_ref, w2b_ref, b2b_ref,
                      wb1_ref, b11_ref, msg_ref, ewb1_ref):
    h1 = jax.nn.relu(g_ref[...].astype(jnp.float32)
                     + ewb_ref[...].astype(jnp.float32))
    h1 = h1.astype(jnp.bfloat16)
    msg_ref[...] = (
        jnp.dot(h1, w2ac_ref[...], preferred_element_type=jnp.float32)
        + cmsg_ref[...])
    e1 = jax.nn.relu(
        jnp.dot(h1, w2b_ref[...], preferred_element_type=jnp.float32)
        + b2b_ref[...])
    ewb1_ref[...] = (
        jnp.dot(e1.astype(jnp.bfloat16), wb1_ref[...],
                preferred_element_type=jnp.float32)
        + b11_ref[...]).astype(jnp.bfloat16)


def _layer0_edge(g0, ewb0, w2ac, cmsg, w2b, b2b, wb1, b11):
    return pl.pallas_call(
        _layer0_edge_body,
        grid=(GE,),
        in_specs=[
            pl.BlockSpec((BE, D), lambda i: (i, 0)),
            pl.BlockSpec((BE, D), lambda i: (i, 0)),
            _full((D, D)),
            _full((1, D)),
            _full((D, D)),
            _full((1, D)),
            _full((D, D)),
            _full((1, D)),
        ],
        out_specs=[
            pl.BlockSpec((BE, D), lambda i: (i, 0)),
            pl.BlockSpec((BE, D), lambda i: (i, 0)),
        ],
        out_shape=[
            jax.ShapeDtypeStruct((E, D), jnp.float32),
            jax.ShapeDtypeStruct((E, D), jnp.bfloat16),
        ],
    )(g0, ewb0, w2ac, cmsg, w2b, b2b, wb1, b11)


def _node_mm_body(x_ref, wa_ref, wc_ref, xa_ref, xc_ref):
    x = x_ref[...].astype(jnp.bfloat16)
    xa_ref[...] = jnp.dot(x, wa_ref[...], preferred_element_type=jnp.float32)
    xc_ref[...] = jnp.dot(x, wc_ref[...], preferred_element_type=jnp.float32)


def _node_mm(x, wa, wc):
    return pl.pallas_call(
        _node_mm_body,
        grid=(GN,),
        in_specs=[
            pl.BlockSpec((BN, D), lambda i: (i, 0)),
            _full((D, D)),
            _full((D, D)),
        ],
        out_specs=[
            pl.BlockSpec((BN, D), lambda i: (i, 0)),
            pl.BlockSpec((BN, D), lambda i: (i, 0)),
        ],
        out_shape=[
            jax.ShapeDtypeStruct((N, D), jnp.float32),
            jax.ShapeDtypeStruct((N, D), jnp.float32),
        ],
    )(x, wa, wc)


def _node_update_body(aggp_ref, degp_ref, x_ref, w1_ref, b1_ref, w2_ref,
                      b2_ref, wa_ref, wc_ref, xa_ref, xc_ref):
    deg = jnp.maximum(degp_ref[0, :, 0:1] + degp_ref[1, :, 0:1], 1.0)
    agg = (aggp_ref[0] + aggp_ref[1]) / deg
    h2 = jax.nn.relu(
        jnp.dot(agg.astype(jnp.bfloat16), w1_ref[...],
                preferred_element_type=jnp.float32)
        + b1_ref[...])
    xn = x_ref[...] + (
        jnp.dot(h2.astype(jnp.bfloat16), w2_ref[...],
                preferred_element_type=jnp.float32)
        + b2_ref[...])
    xn = jax.nn.relu(xn).astype(jnp.bfloat16)
    xa_ref[...] = jnp.dot(xn, wa_ref[...], preferred_element_type=jnp.float32)
    xc_ref[...] = jnp.dot(xn, wc_ref[...], preferred_element_type=jnp.float32)


def _node_update(aggp, degp, x, w1, b1, w2, b2, wa, wc):
    return pl.pallas_call(
        _node_update_body,
        grid=(GN,),
        in_specs=[
            pl.BlockSpec((2, BN, D), lambda i: (0, i, 0)),
            pl.BlockSpec((2, BN, D), lambda i: (0, i, 0)),
            pl.BlockSpec((BN, D), lambda i: (i, 0)),
            _full((D, D)),
            _full((1, D)),
            _full((D, D)),
            _full((1, D)),
            _full((D, D)),
            _full((D, D)),
        ],
        out_specs=[
            pl.BlockSpec((BN, D), lambda i: (i, 0)),
            pl.BlockSpec((BN, D), lambda i: (i, 0)),
        ],
        out_shape=[
            jax.ShapeDtypeStruct((N, D), jnp.float32),
            jax.ShapeDtypeStruct((N, D), jnp.float32),
        ],
    )(aggp, degp, x, w1, b1, w2, b2, wa, wc)


def _layer1_edge_body(g_ref, ewb_ref, w2b_ref, b2b_ref, cw_ref, cb_ref,
                      ef_ref, cpre_ref, ssum_ref, ssq_ref):
    h1 = jax.nn.relu(g_ref[...].astype(jnp.float32)
                     + ewb_ref[...].astype(jnp.float32))
    ef = (jnp.dot(h1.astype(jnp.bfloat16), w2b_ref[...],
                  preferred_element_type=jnp.float32)
          + b2b_ref[...])
    ef_ref[...] = ef
    cpre = (jnp.dot(ef.astype(jnp.bfloat16), cw_ref[...],
                    preferred_element_type=jnp.float32)
            + cb_ref[...])
    cpre_ref[...] = cpre.astype(jnp.bfloat16)

    @pl.when(pl.program_id(0) == 0)
    def _():
        ssum_ref[...] = jnp.zeros_like(ssum_ref)
        ssq_ref[...] = jnp.zeros_like(ssq_ref)

    ssum_ref[...] += jnp.sum(cpre, axis=0, keepdims=True)
    ssq_ref[...] += jnp.sum(cpre * cpre, axis=0, keepdims=True)


def _layer1_edge(g1, ewb1, w2b, b2b, cw1, cb1):
    hc = cw1.shape[1]
    return pl.pallas_call(
        _layer1_edge_body,
        grid=(GE,),
        in_specs=[
            pl.BlockSpec((BE, D), lambda i: (i, 0)),
            pl.BlockSpec((BE, D), lambda i: (i, 0)),
            _full((D, D)),
            _full((1, D)),
            _full((D, hc)),
            _full((1, hc)),
        ],
        out_specs=[
            pl.BlockSpec((BE, D), lambda i: (i, 0)),
            pl.BlockSpec((BE, hc), lambda i: (i, 0)),
            pl.BlockSpec((1, hc), lambda i: (0, 0)),
            pl.BlockSpec((1, hc), lambda i: (0, 0)),
        ],
        out_shape=[
            jax.ShapeDtypeStruct((E, D), jnp.float32),
            jax.ShapeDtypeStruct((E, hc), jnp.bfloat16),
            jax.ShapeDtypeStruct((1, hc), jnp.float32),
            jax.ShapeDtypeStruct((1, hc), jnp.float32),
        ],
    )(g1, ewb1, w2b, b2b, cw1, cb1)


def _cls_finish_body(c_ref, s_ref, t_ref, w_ref, b_ref, out_ref):
    c = jax.nn.relu(c_ref[...].astype(jnp.float32) * s_ref[...] + t_ref[...])
    logits = lax.dot_general(
        w_ref[...], c.astype(jnp.bfloat16), (((1,), (1,)), ((), ())),
        preferred_element_type=jnp.float32) + b_ref[...]
    out_ref[...] = jax.nn.sigmoid(logits)


RP = 32
BEC = 3200
GEC = E // BEC


def _cls_finish(cpre, s, t, wT, bT):
    hc = cpre.shape[1]
    return pl.pallas_call(
        _cls_finish_body,
        grid=(GEC,),
        in_specs=[
            pl.BlockSpec((BEC, hc), lambda i: (i, 0)),
            _full((1, hc)),
            _full((1, hc)),
            _full((RP, hc)),
            _full((RP, 1)),
        ],
        out_specs=pl.BlockSpec((RP, BEC), lambda i: (0, i)),
        out_shape=jax.ShapeDtypeStruct((RP, E), jnp.float32),
    )(cpre, s, t, wT, bT)


def _sc_gather_add(table_a, table_b, idx_ab):
    mesh = plsc.VectorSubcoreMesh(core_axis_name="c", subcore_axis_name="s")

    @functools.partial(
        pl.kernel,
        mesh=mesh,
        out_type=jax.ShapeDtypeStruct((E, D), jnp.float32),
        scratch_types=[
            pltpu.VMEM((2, 2, CH), jnp.int32),
            pltpu.VMEM((2, CH, D), jnp.float32),
            pltpu.VMEM((2, CH, D), jnp.float32),
            pltpu.SemaphoreType.DMA,
            pltpu.SemaphoreType.DMA,
            pltpu.SemaphoreType.DMA,
            pltpu.SemaphoreType.DMA,
            pltpu.SemaphoreType.DMA,
            pltpu.SemaphoreType.DMA,
        ],
    )
    def k(ta_h, tb_h, idx_h, out_h, iv, ra, rb,
          sga0, sga1, sgb0, sgb1, swr0, swr1):
        sga = (sga0, sga1)
        sgb = (sgb0, sgb1)
        swr = (swr0, swr1)
        wid = lax.axis_index("s") * 2 + lax.axis_index("c")
        tw = (NCH - 1 - wid) // NW + 1

        def issue(t, b):
            base = pl.multiple_of((t * NW + wid) * CH, CH)
            pltpu.sync_copy(idx_h.at[:, pl.ds(base, CH)], iv.at[b])
            pltpu.async_copy(ta_h.at[iv.at[b, 0]], ra.at[b], sga[b])
            pltpu.async_copy(tb_h.at[iv.at[b, 1]], rb.at[b], sgb[b])

        for b in (0, 1):
            issue(b, b)

        def pair(g, cc):
            for b in (0, 1):
                t = 2 * g + b

                @pl.when(t < tw)
                def _():
                    base = pl.multiple_of((t * NW + wid) * CH, CH)
                    pltpu.make_async_copy(
                        out_h.at[pl.ds(0, CH)], ra.at[b], sga[b]).wait()
                    pltpu.make_async_copy(
                        out_h.at[pl.ds(0, CH)], rb.at[b], sgb[b]).wait()

                    def addrow(r, cc2):
                        for j in range(D // 16):
                            sl = pl.ds(j * 16, 16)
                            ra[b, r, sl] = ra[b, r, sl] + rb[b, r, sl]
                        return cc2

                    lax.fori_loop(0, CH, addrow, 0)
                    pltpu.async_copy(ra.at[b], out_h.at[pl.ds(base, CH)],
                                     swr[b])

                t2 = t + 2

                @pl.when(t2 < tw)
                def _():
                    pltpu.make_async_copy(
                        out_h.at[pl.ds(0, CH)], ra.at[b], swr[b]).wait()
                    issue(t2, b)
            return cc

        lax.fori_loop(0, (TPW + 1) // 2, pair, 0)

        for b in (0, 1):
            pltpu.make_async_copy(
                out_h.at[pl.ds(0, CH)], ra.at[b], swr[b]).wait()

    return k(table_a, table_b, idx_ab)


def _sc_scatter_msg(msg, dst):
    mesh = plsc.VectorSubcoreMesh(core_axis_name="c", subcore_axis_name="s")

    @functools.partial(
        pl.kernel,
        mesh=mesh,
        out_type=jax.ShapeDtypeStruct((2 * N_PAD, D), jnp.float32),
        scratch_types=[
            pltpu.VMEM((2, CH), jnp.int32),
            pltpu.VMEM((2, CH, D), jnp.float32),
            pltpu.SemaphoreType.DMA,
            pltpu.SemaphoreType.DMA,
            pltpu.VMEM_SHARED((N_PAD, D), jnp.float32),
        ],
    )
    def k(msg_h, dst_h, agg_h, idxv, rows, sm0, sm1, acc_s):
        sm = (sm0, sm1)
        cid = lax.axis_index("c")
        sid = lax.axis_index("s")
        wid = sid * 2 + cid
        tw = (NCH - 1 - wid) // NW + 1

        def zrow(r, cc):
            for j in range(D // 16):
                rows[0, r, pl.ds(j * 16, 16)] = jnp.zeros((16,), jnp.float32)
            return cc

        lax.fori_loop(0, CH, zrow, 0)

        row0 = pl.multiple_of(sid * RPT, CH)
        for off in range(0, RPT, CH):
            pltpu.sync_copy(rows.at[0], acc_s.at[pl.ds(row0 + off, CH)])

        plsc.subcore_barrier()

        def fetch(t, b):
            base = pl.multiple_of((t * NW + wid) * CH, CH)
            pltpu.async_copy(msg_h.at[pl.ds(base, CH)], rows.at[b], sm[b])

        for b in (0, 1):
            fetch(b, b)

        def pair(g, cc):
            for b in (0, 1):
                t = 2 * g + b

                @pl.when(t < tw)
                def _():
                    base = pl.multiple_of((t * NW + wid) * CH, CH)
                    pltpu.sync_copy(dst_h.at[pl.ds(base, CH)], idxv.at[b])
                    pltpu.make_async_copy(
                        msg_h.at[pl.ds(0, CH)], rows.at[b], sm[b]).wait()
                    pltpu.sync_copy(rows.at[b], acc_s.at[idxv.at[b]],
                                    add=True)

                t2 = t + 2

                @pl.when(t2 < tw)
                def _():
                    fetch(t2, b)
            return cc

        lax.fori_loop(0, (TPW + 1) // 2, pair, 0)

        plsc.subcore_barrier()

        out_row0 = pl.multiple_of(cid * N_PAD + sid * RPT, CH)
        for off in range(0, RPT, CH):
            pltpu.sync_copy(acc_s.at[pl.ds(row0 + off, CH)], rows.at[0])
            pltpu.sync_copy(rows.at[0], agg_h.at[pl.ds(out_row0 + off, CH)])

    return k(msg, dst)


def _sc_scatter_ones(dst):
    mesh = plsc.VectorSubcoreMesh(core_axis_name="c", subcore_axis_name="s")

    @functools.partial(
        pl.kernel,
        mesh=mesh,
        out_type=jax.ShapeDtypeStruct((2 * N_PAD, D), jnp.float32),
        scratch_types=[
            pltpu.VMEM((CH,), jnp.int32),
            pltpu.VMEM((CH, D), jnp.float32),
            pltpu.VMEM((CH, D), jnp.float32),
            pltpu.VMEM_SHARED((N_PAD, D), jnp.float32),
        ],
    )
    def k(dst_h, deg_h, idxv, rows, onesv, acc_s):
        cid = lax.axis_index("c")
        sid = lax.axis_index("s")
        wid = sid * 2 + cid

        def fillrow(r, cc):
            for j in range(D // 16):
                rows[r, pl.ds(j * 16, 16)] = jnp.zeros((16,), jnp.float32)
                onesv[r, pl.ds(j * 16, 16)] = jnp.ones((16,), jnp.float32)
            return cc

        lax.fori_loop(0, CH, fillrow, 0)

        row0 = pl.multiple_of(sid * RPT, CH)
        for off in range(0, RPT, CH):
            pltpu.sync_copy(rows, acc_s.at[pl.ds(row0 + off, CH)])

        plsc.subcore_barrier()

        def step(t, carry):
            c = t * NW + wid

            @pl.when(c < NCH)
            def _():
                base = pl.multiple_of(c * CH, CH)
                pltpu.sync_copy(dst_h.at[pl.ds(base, CH)], idxv)
                pltpu.sync_copy(onesv, acc_s.at[idxv], add=True)

            return carry

        lax.fori_loop(0, TPW, step, 0)

        plsc.subcore_barrier()

        out_row0 = pl.multiple_of(cid * N_PAD + sid * RPT, CH)
        for off in range(0, RPT, CH):
            pltpu.sync_copy(acc_s.at[pl.ds(row0 + off, CH)], rows)
            pltpu.sync_copy(rows, deg_h.at[pl.ds(out_row0 + off, CH)])

    return k(dst)


def _bn_fold(ssum, ssq, g, be):
    m = ssum / E
    v = ssq / E - m * m
    s = g / jnp.sqrt(v + 1e-5)
    t = be - m * s
    return s, t


def kernel(obj_feature, rel_feature, edges_index, params):
    p = params
    f32 = jnp.float32
    bf = lambda a: a.astype(jnp.bfloat16)
    row = lambda a: jnp.reshape(a, (1, -1)).astype(f32)

    src = edges_index[0].astype(jnp.int32)
    dst = edges_index[1].astype(jnp.int32)
    idx_ds = jnp.stack([dst, src])

    g0p = p['gcn'][0]
    g1p = p['gcn'][1]
    wa0, wb0, wc0 = (g0p['nn1_w1'][:D], g0p['nn1_w1'][D:2 * D],
                     g0p['nn1_w1'][2 * D:])
    b1_0 = row(g0p['nn1_b1'])
    w2ac0 = g0p['nn1_w2'][:, :D] + g0p['nn1_w2'][:, 2 * D:]
    cmsg0 = row(g0p['nn1_b2'][:D] + g0p['nn1_b2'][2 * D:])
    w2b0 = g0p['nn1_w2'][:, D:2 * D]
    b2b0 = row(g0p['nn1_b2'][D:2 * D])
    wa1, wb1, wc1 = (g1p['nn1_w1'][:D], g1p['nn1_w1'][D:2 * D],
                     g1p['nn1_w1'][2 * D:])
    b1_1 = row(g1p['nn1_b1'])
    w2b1 = g1p['nn1_w2'][:, D:2 * D]
    b2b1 = row(g1p['nn1_b2'][D:2 * D])

    xa0, xc0 = _node_mm(obj_feature, bf(wa0), bf(wc0))
    gsum0 = _sc_gather_add(xa0, xc0, idx_ds)

    y1, ss1, sq1 = _mm_stats(rel_feature, bf(p['enc_w1']), row(p['enc_b1']))
    s1, t1 = _bn_fold(ss1, sq1, row(p['enc_g1']), row(p['enc_be1']))
    y2, ss2, sq2 = _affine_relu_mm_stats(y1, s1, t1, bf(p['enc_w2']),
                                         row(p['enc_b2']))
    s2, t2 = _bn_fold(ss2, sq2, row(p['enc_g2']), row(p['enc_be2']))
    edge_feature, ewb0 = _enc_finish(y2, s2, t2, bf(wb0), b1_0)

    msg0, ewb1 = _layer0_edge(gsum0, ewb0, bf(w2ac0), cmsg0, bf(w2b0), b2b0,
                              bf(wb1), b1_1)

    degp = _sc_scatter_ones(dst)
    aggp = _sc_scatter_msg(msg0, dst)
    xa1, xc1 = _node_update(
        aggp.reshape(2, N_PAD, D), degp.reshape(2, N_PAD, D), obj_feature,
        bf(g0p['nn2_w1']), row(g0p['nn2_b1']), bf(g0p['nn2_w2']),
        row(g0p['nn2_b2']), bf(wa1), bf(wc1))

    gsum1 = _sc_gather_add(xa1, xc1, idx_ds)

    e_final, cpre, ssc, sqc = _layer1_edge(gsum1, ewb1, bf(w2b1), b2b1,
                                           bf(p['cls_w1']), row(p['cls_b1']))
    s_c, t_c = _bn_fold(ssc, sqc, row(p['cls_g1']), row(p['cls_be1']))
    w2t = jnp.zeros((RP, p['cls_w2'].shape[0]), jnp.bfloat16)
    w2t = w2t.at[:R].set(bf(p['cls_w2'].T))
    b2t = jnp.zeros((RP, 1), f32).at[:R].set(
        jnp.reshape(p['cls_b2'], (-1, 1)).astype(f32))
    rel_cls = _cls_finish(cpre, s_c, t_c, w2t, b2t)[:R].T

    return rel_cls, obj_feature, edge_feature, e_final


# speedup vs baseline: 4.0950x; 1.0084x over previous
import functools

import jax
import jax.numpy as jnp
from jax import lax
from jax.experimental import pallas as pl
from jax.experimental.pallas import tpu as pltpu
from jax.experimental.pallas import tpu_sc as plsc

N = 10000
E = 160000
D = 128
R = 26

BE = 16000
GE = E // BE
BN = 2000
GN = N // BN

CH = 128
NCH = E // CH
NW = 32
TPW = (NCH + NW - 1) // NW
NS = 16
RPT = 640
N_PAD = NS * RPT


def _full(shape):
    return pl.BlockSpec(shape, lambda i: tuple(0 for _ in shape))


def _mm_stats_body(x_ref, w_ref, b_ref, y_ref, ssum_ref, ssq_ref):
    y = jnp.dot(x_ref[...].astype(jnp.bfloat16), w_ref[...],
                preferred_element_type=jnp.float32)
    y = y + b_ref[...]
    y_ref[...] = y.astype(jnp.bfloat16)

    @pl.when(pl.program_id(0) == 0)
    def _():
        ssum_ref[...] = jnp.zeros_like(ssum_ref)
        ssq_ref[...] = jnp.zeros_like(ssq_ref)

    ssum_ref[...] += jnp.sum(y, axis=0, keepdims=True)
    ssq_ref[...] += jnp.sum(y * y, axis=0, keepdims=True)


def _mm_stats(x, w, b):
    k_in = x.shape[1]
    k_out = w.shape[1]
    return pl.pallas_call(
        _mm_stats_body,
        grid=(GE,),
        in_specs=[
            pl.BlockSpec((BE, k_in), lambda i: (i, 0)),
            _full((k_in, k_out)),
            _full((1, k_out)),
        ],
        out_specs=[
            pl.BlockSpec((BE, k_out), lambda i: (i, 0)),
            pl.BlockSpec((1, k_out), lambda i: (0, 0)),
            pl.BlockSpec((1, k_out), lambda i: (0, 0)),
        ],
        out_shape=[
            jax.ShapeDtypeStruct((E, k_out), jnp.bfloat16),
            jax.ShapeDtypeStruct((1, k_out), jnp.float32),
            jax.ShapeDtypeStruct((1, k_out), jnp.float32),
        ],
    )(x, w, b)


def _affine_relu_mm_stats_body(x_ref, s_ref, t_ref, w_ref, b_ref,
                               y_ref, ssum_ref, ssq_ref):
    h = jax.nn.relu(x_ref[...].astype(jnp.float32) * s_ref[...] + t_ref[...])
    y = jnp.dot(h.astype(jnp.bfloat16), w_ref[...],
                preferred_element_type=jnp.float32) + b_ref[...]
    y_ref[...] = y.astype(jnp.bfloat16)

    @pl.when(pl.program_id(0) == 0)
    def _():
        ssum_ref[...] = jnp.zeros_like(ssum_ref)
        ssq_ref[...] = jnp.zeros_like(ssq_ref)

    ssum_ref[...] += jnp.sum(y, axis=0, keepdims=True)
    ssq_ref[...] += jnp.sum(y * y, axis=0, keepdims=True)


def _affine_relu_mm_stats(x, s, t, w, b):
    k_in = x.shape[1]
    k_out = w.shape[1]
    return pl.pallas_call(
        _affine_relu_mm_stats_body,
        grid=(GE,),
        in_specs=[
            pl.BlockSpec((BE, k_in), lambda i: (i, 0)),
            _full((1, k_in)),
            _full((1, k_in)),
            _full((k_in, k_out)),
            _full((1, k_out)),
        ],
        out_specs=[
            pl.BlockSpec((BE, k_out), lambda i: (i, 0)),
            pl.BlockSpec((1, k_out), lambda i: (0, 0)),
            pl.BlockSpec((1, k_out), lambda i: (0, 0)),
        ],
        out_shape=[
            jax.ShapeDtypeStruct((E, k_out), jnp.bfloat16),
            jax.ShapeDtypeStruct((1, k_out), jnp.float32),
            jax.ShapeDtypeStruct((1, k_out), jnp.float32),
        ],
    )(x, s, t, w, b)


def _enc_finish_body(y_ref, s_ref, t_ref, w_ref, b_ref, ef_ref, ewb_ref):
    ef = jax.nn.relu(y_ref[...].astype(jnp.float32) * s_ref[...] + t_ref[...])
    ef_ref[...] = ef
    ewb_ref[...] = (
        jnp.dot(ef.astype(jnp.bfloat16), w_ref[...],
                preferred_element_type=jnp.float32)
        + b_ref[...]).astype(jnp.bfloat16)


def _enc_finish(y2, s, t, wb0, b1_0):
    return pl.pallas_call(
        _enc_finish_body,
        grid=(GE,),
        in_specs=[
            pl.BlockSpec((BE, D), lambda i: (i, 0)),
            _full((1, D)),
            _full((1, D)),
            _full((D, D)),
            _full((1, D)),
        ],
        out_specs=[
            pl.BlockSpec((BE, D), lambda i: (i, 0)),
            pl.BlockSpec((BE, D), lambda i: (i, 0)),
        ],
        out_shape=[
            jax.ShapeDtypeStruct((E, D), jnp.float32),
            jax.ShapeDtypeStruct((E, D), jnp.bfloat16),
        ],
    )(y2, s, t, wb0, b1_0)


def _layer0_edge_body(g_ref, ewb_ref, w2ac_ref, cmsg_ref, w2b_ref, b2b_ref,
                      wb1_ref, b11_ref, msg_ref, ewb1_ref):
    h1 = jax.nn.relu(g_ref[...].astype(jnp.float32)
                     + ewb_ref[...].astype(jnp.float32))
    h1 = h1.astype(jnp.bfloat16)
    msg_ref[...] = (
        jnp.dot(h1, w2ac_ref[...], preferred_element_type=jnp.float32)
        + cmsg_ref[...])
    e1 = jax.nn.relu(
        jnp.dot(h1, w2b_ref[...], preferred_element_type=jnp.float32)
        + b2b_ref[...])
    ewb1_ref[...] = (
        jnp.dot(e1.astype(jnp.bfloat16), wb1_ref[...],
                preferred_element_type=jnp.float32)
        + b11_ref[...]).astype(jnp.bfloat16)


def _layer0_edge(g0, ewb0, w2ac, cmsg, w2b, b2b, wb1, b11):
    return pl.pallas_call(
        _layer0_edge_body,
        grid=(GE,),
        in_specs=[
            pl.BlockSpec((BE, D), lambda i: (i, 0)),
            pl.BlockSpec((BE, D), lambda i: (i, 0)),
            _full((D, D)),
            _full((1, D)),
            _full((D, D)),
            _full((1, D)),
            _full((D, D)),
            _full((1, D)),
        ],
        out_specs=[
            pl.BlockSpec((BE, D), lambda i: (i, 0)),
            pl.BlockSpec((BE, D), lambda i: (i, 0)),
        ],
        out_shape=[
            jax.ShapeDtypeStruct((E, D), jnp.float32),
            jax.ShapeDtypeStruct((E, D), jnp.bfloat16),
        ],
    )(g0, ewb0, w2ac, cmsg, w2b, b2b, wb1, b11)


def _node_mm_body(x_ref, wa_ref, wc_ref, xa_ref, xc_ref):
    x = x_ref[...].astype(jnp.bfloat16)
    xa_ref[...] = jnp.dot(x, wa_ref[...], preferred_element_type=jnp.float32)
    xc_ref[...] = jnp.dot(x, wc_ref[...], preferred_element_type=jnp.float32)


def _node_mm(x, wa, wc):
    return pl.pallas_call(
        _node_mm_body,
        grid=(GN,),
        in_specs=[
            pl.BlockSpec((BN, D), lambda i: (i, 0)),
            _full((D, D)),
            _full((D, D)),
        ],
        out_specs=[
            pl.BlockSpec((BN, D), lambda i: (i, 0)),
            pl.BlockSpec((BN, D), lambda i: (i, 0)),
        ],
        out_shape=[
            jax.ShapeDtypeStruct((N, D), jnp.float32),
            jax.ShapeDtypeStruct((N, D), jnp.float32),
        ],
    )(x, wa, wc)


def _node_update_body(aggp_ref, degp_ref, x_ref, w1_ref, b1_ref, w2_ref,
                      b2_ref, wa_ref, wc_ref, xa_ref, xc_ref):
    deg = jnp.maximum(degp_ref[0, :, 0:1] + degp_ref[1, :, 0:1], 1.0)
    agg = (aggp_ref[0] + aggp_ref[1]) / deg
    h2 = jax.nn.relu(
        jnp.dot(agg.astype(jnp.bfloat16), w1_ref[...],
                preferred_element_type=jnp.float32)
        + b1_ref[...])
    xn = x_ref[...] + (
        jnp.dot(h2.astype(jnp.bfloat16), w2_ref[...],
                preferred_element_type=jnp.float32)
        + b2_ref[...])
    xn = jax.nn.relu(xn).astype(jnp.bfloat16)
    xa_ref[...] = jnp.dot(xn, wa_ref[...], preferred_element_type=jnp.float32)
    xc_ref[...] = jnp.dot(xn, wc_ref[...], preferred_element_type=jnp.float32)


def _node_update(aggp, degp, x, w1, b1, w2, b2, wa, wc):
    return pl.pallas_call(
        _node_update_body,
        grid=(GN,),
        in_specs=[
            pl.BlockSpec((2, BN, D), lambda i: (0, i, 0)),
            pl.BlockSpec((2, BN, D), lambda i: (0, i, 0)),
            pl.BlockSpec((BN, D), lambda i: (i, 0)),
            _full((D, D)),
            _full((1, D)),
            _full((D, D)),
            _full((1, D)),
            _full((D, D)),
            _full((D, D)),
        ],
        out_specs=[
            pl.BlockSpec((BN, D), lambda i: (i, 0)),
            pl.BlockSpec((BN, D), lambda i: (i, 0)),
        ],
        out_shape=[
            jax.ShapeDtypeStruct((N, D), jnp.float32),
            jax.ShapeDtypeStruct((N, D), jnp.float32),
        ],
    )(aggp, degp, x, w1, b1, w2, b2, wa, wc)


def _layer1_edge_body(g_ref, ewb_ref, w2b_ref, b2b_ref, cw_ref, cb_ref,
                      ef_ref, cpre_ref, ssum_ref, ssq_ref):
    h1 = jax.nn.relu(g_ref[...].astype(jnp.float32)
                     + ewb_ref[...].astype(jnp.float32))
    ef = (jnp.dot(h1.astype(jnp.bfloat16), w2b_ref[...],
                  preferred_element_type=jnp.float32)
          + b2b_ref[...])
    ef_ref[...] = ef
    cpre = (jnp.dot(ef.astype(jnp.bfloat16), cw_ref[...],
                    preferred_element_type=jnp.float32)
            + cb_ref[...])
    cpre_ref[...] = cpre.astype(jnp.bfloat16)

    @pl.when(pl.program_id(0) == 0)
    def _():
        ssum_ref[...] = jnp.zeros_like(ssum_ref)
        ssq_ref[...] = jnp.zeros_like(ssq_ref)

    ssum_ref[...] += jnp.sum(cpre, axis=0, keepdims=True)
    ssq_ref[...] += jnp.sum(cpre * cpre, axis=0, keepdims=True)


def _layer1_edge(g1, ewb1, w2b, b2b, cw1, cb1):
    hc = cw1.shape[1]
    return pl.pallas_call(
        _layer1_edge_body,
        grid=(GE,),
        in_specs=[
            pl.BlockSpec((BE, D), lambda i: (i, 0)),
            pl.BlockSpec((BE, D), lambda i: (i, 0)),
            _full((D, D)),
            _full((1, D)),
            _full((D, hc)),
            _full((1, hc)),
        ],
        out_specs=[
            pl.BlockSpec((BE, D), lambda i: (i, 0)),
            pl.BlockSpec((BE, hc), lambda i: (i, 0)),
            pl.BlockSpec((1, hc), lambda i: (0, 0)),
            pl.BlockSpec((1, hc), lambda i: (0, 0)),
        ],
        out_shape=[
            jax.ShapeDtypeStruct((E, D), jnp.float32),
            jax.ShapeDtypeStruct((E, hc), jnp.bfloat16),
            jax.ShapeDtypeStruct((1, hc), jnp.float32),
            jax.ShapeDtypeStruct((1, hc), jnp.float32),
        ],
    )(g1, ewb1, w2b, b2b, cw1, cb1)


def _cls_finish_body(c_ref, s_ref, t_ref, w_ref, b_ref, out_ref):
    c = jax.nn.relu(c_ref[...].astype(jnp.float32) * s_ref[...] + t_ref[...])
    logits = lax.dot_general(
        w_ref[...], c.astype(jnp.bfloat16), (((1,), (1,)), ((), ())),
        preferred_element_type=jnp.float32) + b_ref[...]
    out_ref[...] = jax.nn.sigmoid(logits)


RP = 32
BEC = 3200
GEC = E // BEC


def _cls_finish(cpre, s, t, wT, bT):
    hc = cpre.shape[1]
    return pl.pallas_call(
        _cls_finish_body,
        grid=(GEC,),
        in_specs=[
            pl.BlockSpec((BEC, hc), lambda i: (i, 0)),
            _full((1, hc)),
            _full((1, hc)),
            _full((RP, hc)),
            _full((RP, 1)),
        ],
        out_specs=pl.BlockSpec((RP, BEC), lambda i: (0, i)),
        out_shape=jax.ShapeDtypeStruct((RP, E), jnp.float32),
    )(cpre, s, t, wT, bT)


def _sc_gather_add(table_a, table_b, idx_ab):
    mesh = plsc.VectorSubcoreMesh(core_axis_name="c", subcore_axis_name="s")

    @functools.partial(
        pl.kernel,
        mesh=mesh,
        out_type=jax.ShapeDtypeStruct((E, D), jnp.float32),
        scratch_types=[
            pltpu.VMEM((2, 2, CH), jnp.int32),
            pltpu.VMEM((2, CH, D), jnp.float32),
            pltpu.VMEM((2, CH, D), jnp.float32),
            pltpu.SemaphoreType.DMA,
            pltpu.SemaphoreType.DMA,
            pltpu.SemaphoreType.DMA,
            pltpu.SemaphoreType.DMA,
            pltpu.SemaphoreType.DMA,
            pltpu.SemaphoreType.DMA,
        ],
    )
    def k(ta_h, tb_h, idx_h, out_h, iv, ra, rb,
          sga0, sga1, sgb0, sgb1, swr0, swr1):
        sga = (sga0, sga1)
        sgb = (sgb0, sgb1)
        swr = (swr0, swr1)
        wid = lax.axis_index("s") * 2 + lax.axis_index("c")
        tw = (NCH - 1 - wid) // NW + 1

        def issue(t, b):
            base = pl.multiple_of((t * NW + wid) * CH, CH)
            pltpu.sync_copy(idx_h.at[:, pl.ds(base, CH)], iv.at[b])
            pltpu.async_copy(ta_h.at[iv.at[b, 0]], ra.at[b], sga[b])
            pltpu.async_copy(tb_h.at[iv.at[b, 1]], rb.at[b], sgb[b])

        for b in (0, 1):
            issue(b, b)

        def pair(g, cc):
            for b in (0, 1):
                t = 2 * g + b

                @pl.when(t < tw)
                def _():
                    base = pl.multiple_of((t * NW + wid) * CH, CH)
                    pltpu.make_async_copy(
                        out_h.at[pl.ds(0, CH)], ra.at[b], sga[b]).wait()
                    pltpu.make_async_copy(
                        out_h.at[pl.ds(0, CH)], rb.at[b], sgb[b]).wait()

                    def addrow(r, cc2):
                        for j in range(D // 16):
                            sl = pl.ds(j * 16, 16)
                            ra[b, r, sl] = ra[b, r, sl] + rb[b, r, sl]
                        return cc2

                    lax.fori_loop(0, CH, addrow, 0)
                    pltpu.async_copy(ra.at[b], out_h.at[pl.ds(base, CH)],
                                     swr[b])

                t2 = t + 2

                @pl.when(t2 < tw)
                def _():
                    pltpu.make_async_copy(
                        out_h.at[pl.ds(0, CH)], ra.at[b], swr[b]).wait()
                    issue(t2, b)
            return cc

        lax.fori_loop(0, (TPW + 1) // 2, pair, 0)

        for b in (0, 1):
            pltpu.make_async_copy(
                out_h.at[pl.ds(0, CH)], ra.at[b], swr[b]).wait()

    return k(table_a, table_b, idx_ab)


def _sc_scatter_msg(msg, dst):
    mesh = plsc.VectorSubcoreMesh(core_axis_name="c", subcore_axis_name="s")

    @functools.partial(
        pl.kernel,
        mesh=mesh,
        out_type=jax.ShapeDtypeStruct((2 * N_PAD, D), jnp.float32),
        scratch_types=[
            pltpu.VMEM((2, CH), jnp.int32),
            pltpu.VMEM((2, CH, D), jnp.float32),
            pltpu.SemaphoreType.DMA,
            pltpu.SemaphoreType.DMA,
            pltpu.VMEM_SHARED((N_PAD, D), jnp.float32),
        ],
    )
    def k(msg_h, dst_h, agg_h, idxv, rows, sm0, sm1, acc_s):
        sm = (sm0, sm1)
        cid = lax.axis_index("c")
        sid = lax.axis_index("s")
        wid = sid * 2 + cid
        tw = (NCH - 1 - wid) // NW + 1

        def zrow(r, cc):
            for j in range(D // 16):
                rows[0, r, pl.ds(j * 16, 16)] = jnp.zeros((16,), jnp.float32)
            return cc

        lax.fori_loop(0, CH, zrow, 0)

        row0 = pl.multiple_of(sid * RPT, CH)
        for off in range(0, RPT, CH):
            pltpu.sync_copy(rows.at[0], acc_s.at[pl.ds(row0 + off, CH)])

        plsc.subcore_barrier()

        def fetch(t, b):
            base = pl.multiple_of((t * NW + wid) * CH, CH)
            pltpu.async_copy(msg_h.at[pl.ds(base, CH)], rows.at[b], sm[b])

        for b in (0, 1):
            fetch(b, b)

        def pair(g, cc):
            for b in (0, 1):
                t = 2 * g + b

                @pl.when(t < tw)
                def _():
                    base = pl.multiple_of((t * NW + wid) * CH, CH)
                    pltpu.sync_copy(dst_h.at[pl.ds(base, CH)], idxv.at[b])
                    pltpu.make_async_copy(
                        msg_h.at[pl.ds(0, CH)], rows.at[b], sm[b]).wait()
                    pltpu.sync_copy(rows.at[b], acc_s.at[idxv.at[b]],
                                    add=True)

                t2 = t + 2

                @pl.when(t2 < tw)
                def _():
                    fetch(t2, b)
            return cc

        lax.fori_loop(0, (TPW + 1) // 2, pair, 0)

        plsc.subcore_barrier()

        out_row0 = pl.multiple_of(cid * N_PAD + sid * RPT, CH)
        for off in range(0, RPT, CH):
            pltpu.sync_copy(acc_s.at[pl.ds(row0 + off, CH)], rows.at[0])
            pltpu.sync_copy(rows.at[0], agg_h.at[pl.ds(out_row0 + off, CH)])

    return k(msg, dst)


def _sc_scatter_ones(dst):
    mesh = plsc.VectorSubcoreMesh(core_axis_name="c", subcore_axis_name="s")

    @functools.partial(
        pl.kernel,
        mesh=mesh,
        out_type=jax.ShapeDtypeStruct((2 * N_PAD, D), jnp.float32),
        scratch_types=[
            pltpu.VMEM((CH,), jnp.int32),
            pltpu.VMEM((CH, D), jnp.float32),
            pltpu.VMEM((CH, D), jnp.float32),
            pltpu.VMEM_SHARED((N_PAD, D), jnp.float32),
        ],
    )
    def k(dst_h, deg_h, idxv, rows, onesv, acc_s):
        cid = lax.axis_index("c")
        sid = lax.axis_index("s")
        wid = sid * 2 + cid

        def fillrow(r, cc):
            for j in range(D // 16):
                rows[r, pl.ds(j * 16, 16)] = jnp.zeros((16,), jnp.float32)
                onesv[r, pl.ds(j * 16, 16)] = jnp.ones((16,), jnp.float32)
            return cc

        lax.fori_loop(0, CH, fillrow, 0)

        row0 = pl.multiple_of(sid * RPT, CH)
        for off in range(0, RPT, CH):
            pltpu.sync_copy(rows, acc_s.at[pl.ds(row0 + off, CH)])

        plsc.subcore_barrier()

        def step(t, carry):
            c = t * NW + wid

            @pl.when(c < NCH)
            def _():
                base = pl.multiple_of(c * CH, CH)
                pltpu.sync_copy(dst_h.at[pl.ds(base, CH)], idxv)
                pltpu.sync_copy(onesv, acc_s.at[idxv], add=True)

            return carry

        lax.fori_loop(0, TPW, step, 0)

        plsc.subcore_barrier()

        out_row0 = pl.multiple_of(cid * N_PAD + sid * RPT, CH)
        for off in range(0, RPT, CH):
            pltpu.sync_copy(acc_s.at[pl.ds(row0 + off, CH)], rows)
            pltpu.sync_copy(rows, deg_h.at[pl.ds(out_row0 + off, CH)])

    return k(dst)


def _bn_fold(ssum, ssq, g, be):
    m = ssum / E
    v = ssq / E - m * m
    s = g / jnp.sqrt(v + 1e-5)
    t = be - m * s
    return s, t


def kernel(obj_feature, rel_feature, edges_index, params):
    p = params
    f32 = jnp.float32
    bf = lambda a: a.astype(jnp.bfloat16)
    row = lambda a: jnp.reshape(a, (1, -1)).astype(f32)

    src = edges_index[0].astype(jnp.int32)
    dst = edges_index[1].astype(jnp.int32)
    idx_ds = jnp.stack([dst, src])

    g0p = p['gcn'][0]
    g1p = p['gcn'][1]
    wa0, wb0, wc0 = (g0p['nn1_w1'][:D], g0p['nn1_w1'][D:2 * D],
                     g0p['nn1_w1'][2 * D:])
    b1_0 = row(g0p['nn1_b1'])
    w2ac0 = g0p['nn1_w2'][:, :D] + g0p['nn1_w2'][:, 2 * D:]
    cmsg0 = row(g0p['nn1_b2'][:D] + g0p['nn1_b2'][2 * D:])
    w2b0 = g0p['nn1_w2'][:, D:2 * D]
    b2b0 = row(g0p['nn1_b2'][D:2 * D])
    wa1, wb1, wc1 = (g1p['nn1_w1'][:D], g1p['nn1_w1'][D:2 * D],
                     g1p['nn1_w1'][2 * D:])
    b1_1 = row(g1p['nn1_b1'])
    w2b1 = g1p['nn1_w2'][:, D:2 * D]
    b2b1 = row(g1p['nn1_b2'][D:2 * D])

    xa0, xc0 = _node_mm(obj_feature, bf(wa0), bf(wc0))
    gsum0 = _sc_gather_add(xa0, xc0, idx_ds)

    y1, ss1, sq1 = _mm_stats(rel_feature, bf(p['enc_w1']), row(p['enc_b1']))
    s1, t1 = _bn_fold(ss1, sq1, row(p['enc_g1']), row(p['enc_be1']))
    y2, ss2, sq2 = _affine_relu_mm_stats(y1, s1, t1, bf(p['enc_w2']),
                                         row(p['enc_b2']))
    s2, t2 = _bn_fold(ss2, sq2, row(p['enc_g2']), row(p['enc_be2']))
    edge_feature, ewb0 = _enc_finish(y2, s2, t2, bf(wb0), b1_0)

    msg0, ewb1 = _layer0_edge(gsum0, ewb0, bf(w2ac0), cmsg0, bf(w2b0), b2b0,
                              bf(wb1), b1_1)

    degp = _sc_scatter_ones(dst)
    aggp = _sc_scatter_msg(msg0, dst)
    xa1, xc1 = _node_update(
        aggp.reshape(2, N_PAD, D), degp.reshape(2, N_PAD, D), obj_feature,
        bf(g0p['nn2_w1']), row(g0p['nn2_b1']), bf(g0p['nn2_w2']),
        row(g0p['nn2_b2']), bf(wa1), bf(wc1))

    gsum1 = _sc_gather_add(xa1, xc1, idx_ds)

    e_final, cpre, ssc, sqc = _layer1_edge(gsum1, ewb1, bf(w2b1), b2b1,
                                           bf(p['cls_w1']), row(p['cls_b1']))
    s_c, t_c = _bn_fold(ssc, sqc, row(p['cls_g1']), row(p['cls_be1']))
    w2t = jnp.zeros((RP, p['cls_w2'].shape[0]), jnp.bfloat16)
    w2t = w2t.at[:R].set(bf(p['cls_w2'].T))
    b2t = jnp.zeros((RP, 1), f32).at[:R].set(
        jnp.reshape(p['cls_b2'], (-1, 1)).astype(f32))
    rel_cls = _cls_finish(cpre, s_c, t_c, w2t, b2t)[:R].T

    return rel_cls, obj_feature, edge_feature, e_final
